# Initial kernel scaffold; baseline (speedup 1.0000x reference)
import jax
import jax.numpy as jnp
from jax.experimental import pallas as pl


def kernel(x, pos, edge_index, W1, b1, gamma1, beta1, W2, b2, gamma2, beta2, W3, b3, gamma3, beta3, W4, b4, gamma4, beta4, W5, b5, gamma5, beta5, W6, b6, gamma6, beta6, W7, b7, gamma7, beta7, fcW):
    raise NotImplementedError("write your pallas kernel here")


# speedup vs baseline: 27.9865x; 27.9865x over previous
import functools

import jax
import jax.numpy as jnp
from jax import lax
from jax.experimental import pallas as pl
from jax.experimental.pallas import tpu as pltpu
from jax.experimental.pallas import tpu_sc as plsc

N = 100000
E = 1600000
PX, PY = 16.0, 12.0
NX, NY = 15, 15
NC1 = NX * NY
HP = 232
P7X, P7Y = 60.0, 45.0
NC7 = 16
EPS = 1e-5

BR = 2000
GRID = N // BR


def _cl_body(pos_ref, cl_ref):
    p = pos_ref[...]
    cx = jnp.clip(jnp.floor(p[:, 0:1] / PX), 0, NX - 1)
    cy = jnp.clip(jnp.floor(p[:, 1:2] / PY), 0, NY - 1)
    cl_ref[...] = (cx * NY + cy).astype(jnp.int32)


def _cl_from_pos(pos):
    return pl.pallas_call(
        _cl_body,
        grid=(GRID,),
        in_specs=[pl.BlockSpec((BR, 2), lambda i: (i, 0))],
        out_specs=pl.BlockSpec((BR, 1), lambda i: (i, 0)),
        out_shape=jax.ShapeDtypeStruct((N, 1), jnp.int32),
    )(pos)


def _tc1_body(d0_ref, d1_ref, x_ref, w1_ref, dinv_ref, z1_ref):
    deg = d0_ref[...] + d1_ref[...] + 1.0
    dinv = lax.rsqrt(deg)
    dinv_ref[...] = dinv
    z1_ref[...] = (x_ref[...] * dinv) * w1_ref[...]


def _tc1(deg0, deg1, x, w1p):
    return pl.pallas_call(
        _tc1_body,
        grid=(GRID,),
        in_specs=[
            pl.BlockSpec((BR, 1), lambda i: (i, 0)),
            pl.BlockSpec((BR, 1), lambda i: (i, 0)),
            pl.BlockSpec((BR, 1), lambda i: (i, 0)),
            pl.BlockSpec((1, 16), lambda i: (0, 0)),
        ],
        out_specs=[
            pl.BlockSpec((BR, 1), lambda i: (i, 0)),
            pl.BlockSpec((BR, 16), lambda i: (i, 0)),
        ],
        out_shape=[
            jax.ShapeDtypeStruct((N, 1), jnp.float32),
            jax.ShapeDtypeStruct((N, 16), jnp.float32),
        ],
    )(deg0, deg1, x, w1p)


def _make_stats(nparts):
    C = 16 * nparts

    def body(*refs):
        part_refs = refs[:2 * nparts]
        z_ref, dinv_ref, p_ref, stats_ref = refs[2 * nparts:]
        agg = jnp.concatenate(
            [part_refs[2 * i][...] + part_refs[2 * i + 1][...]
             for i in range(nparts)], axis=1)
        p = dinv_ref[...] * (agg + z_ref[...])
        p_ref[...] = p
        s1 = jnp.sum(p, axis=0, keepdims=True)
        s2 = jnp.sum(p * p, axis=0, keepdims=True)
        upd = jnp.concatenate([s1, s2, jnp.zeros((6, C), jnp.float32)], axis=0)

        @pl.when(pl.program_id(0) == 0)
        def _():
            stats_ref[...] = jnp.zeros_like(stats_ref)

        stats_ref[...] += upd

    def call(parts, z, dinv):
        return pl.pallas_call(
            body,
            grid=(GRID,),
            in_specs=[pl.BlockSpec((BR, 16), lambda i: (i, 0))] * (2 * nparts)
            + [
                pl.BlockSpec((BR, C), lambda i: (i, 0)),
                pl.BlockSpec((BR, 1), lambda i: (i, 0)),
            ],
            out_specs=[
                pl.BlockSpec((BR, C), lambda i: (i, 0)),
                pl.BlockSpec((8, C), lambda i: (0, 0)),
            ],
            out_shape=[
                jax.ShapeDtypeStruct((N, C), jnp.float32),
                jax.ShapeDtypeStruct((8, C), jnp.float32),
            ],
        )(*parts, z, dinv)

    return call


def _make_epilogue(Cp, Cin, Cout, residual, matmul):
    def body(*refs):
        it = iter(refs)
        p_ref = next(it)
        stats_ref = next(it)
        g_ref = next(it)
        b_ref = next(it)
        w_ref = next(it) if matmul else None
        dinv_ref = next(it) if matmul else None
        res_ref = next(it) if residual else None
        x_ref = next(it)
        z_ref = next(it) if matmul else None

        p = p_ref[...][:, :Cin]
        mean = stats_ref[0:1, :Cin] / N
        var = stats_ref[1:2, :Cin] / N - mean * mean
        x = (p - mean) * lax.rsqrt(var + EPS) * g_ref[...] + b_ref[...]
        x = jnp.maximum(x, 0.0)
        if residual:
            x = x + res_ref[...]
        x_ref[...] = x
        if matmul:
            z_ref[...] = (
                jnp.dot(x, w_ref[...], preferred_element_type=jnp.float32)
                * dinv_ref[...]
            )

    def call(p, stats, gamma, beta, W=None, dinv=None, res=None):
        in_specs = [
            pl.BlockSpec((BR, Cp), lambda i: (i, 0)),
            pl.BlockSpec((8, Cp), lambda i: (0, 0)),
            pl.BlockSpec((1, Cin), lambda i: (0, 0)),
            pl.BlockSpec((1, Cin), lambda i: (0, 0)),
        ]
        args = [p, stats, gamma, beta]
        if matmul:
            in_specs.append(pl.BlockSpec((Cin, Cout), lambda i: (0, 0)))
            in_specs.append(pl.BlockSpec((BR, 1), lambda i: (i, 0)))
            args += [W, dinv]
        if residual:
            in_specs.append(pl.BlockSpec((BR, Cin), lambda i: (i, 0)))
            args.append(res)
        out_specs = [pl.BlockSpec((BR, Cin), lambda i: (i, 0))]
        out_shape = [jax.ShapeDtypeStruct((N, Cin), jnp.float32)]
        if matmul:
            out_specs.append(pl.BlockSpec((BR, Cout), lambda i: (i, 0)))
            out_shape.append(jax.ShapeDtypeStruct((N, Cout), jnp.float32))
        res_ = pl.pallas_call(
            body,
            grid=(GRID,),
            in_specs=in_specs,
            out_specs=out_specs,
            out_shape=out_shape,
        )(*args)
        return res_ if matmul else res_[0]

    return call


def _final_body(h0_ref, h1_ref, tbl_ref, w6_ref, g6_ref, b6_ref, w7_ref,
                g7_ref, b7_ref, fcw_ref, out_ref):
    cm = h0_ref[...] + h1_ref[...]
    cm = cm[:, :NC1]
    rows = lax.broadcasted_iota(jnp.int32, (NC1, NC1), 0)
    cols = lax.broadcasted_iota(jnp.int32, (NC1, NC1), 1)
    cm = cm + jnp.where(rows == cols, 1.0, 0.0)
    degp = jnp.sum(cm, axis=1, keepdims=True)
    dp = lax.rsqrt(degp)

    xp = jnp.max(tbl_ref[...], axis=0)
    xp = jnp.where(jnp.isfinite(xp), xp, 0.0)

    def bn_relu(p, g, b):
        mean = jnp.mean(p, axis=0, keepdims=True)
        var = jnp.mean(p * p, axis=0, keepdims=True) - mean * mean
        return jnp.maximum((p - mean) * lax.rsqrt(var + EPS) * g + b, 0.0)

    y = jnp.dot(xp, w6_ref[...], preferred_element_type=jnp.float32)
    p = dp * jnp.dot(cm, dp * y, preferred_element_type=jnp.float32)
    x = bn_relu(p, g6_ref[...], b6_ref[...])
    y = jnp.dot(x, w7_ref[...], preferred_element_type=jnp.float32)
    p = dp * jnp.dot(cm, dp * y, preferred_element_type=jnp.float32)
    x = bn_relu(p, g7_ref[...], b7_ref[...]) + xp

    ids = lax.broadcasted_iota(jnp.int32, (NC1, 1), 0)
    r = ids // NY
    c = ids % NY
    c7x = jnp.clip(jnp.floor((r.astype(jnp.float32) + 0.5) * PX / P7X), 0, 3)
    c7y = jnp.clip(jnp.floor((c.astype(jnp.float32) + 0.5) * PY / P7Y), 0, 3)
    c7 = (c7x * 4 + c7y).astype(jnp.int32)
    acc = jnp.zeros((1, 2), jnp.float32)
    for k in range(NC7):
        mk = jnp.max(jnp.where(c7 == k, x, -jnp.inf), axis=0, keepdims=True)
        mk = jnp.where(jnp.isfinite(mk), mk, 0.0)
        acc = acc + jnp.dot(mk, fcw_ref[k], preferred_element_type=jnp.float32)
    out_ref[...] = acc


def _final(h0, h1, tables, W6, g6, b6, W7, g7, b7, fcw3):
    return pl.pallas_call(
        _final_body,
        out_shape=jax.ShapeDtypeStruct((1, 2), jnp.float32),
    )(h0, h1, tables, W6, g6, b6, W7, g7, b7, fcw3)


_SC_MESH = plsc.VectorSubcoreMesh(core_axis_name="c", subcore_axis_name="s")
_SC_PARAMS = pltpu.CompilerParams(use_tc_tiling_on_sc=False)
_NCORE, _NSUB, _NW = 2, 16, 32
_B = 128
_ROWS = E // _B
_RPW = _ROWS // _NW
_LEFT = _ROWS - _NW * _RPW
_NT = 6144
_NEXTRA = N - _NSUB * _NT
_HISTP = 52224
_HT = 3200
_HEXTRA = _HISTP - _NSUB * _HT


def _sc_aggregate(z, src1, dst1, zeros2d):
    @functools.partial(
        pl.kernel,
        out_type=[
            jax.ShapeDtypeStruct((N, 16), jnp.float32),
            jax.ShapeDtypeStruct((N, 16), jnp.float32),
        ],
        mesh=_SC_MESH,
        compiler_params=_SC_PARAMS,
        scratch_types=[
            pltpu.VMEM((_B,), jnp.int32),
            pltpu.VMEM((_B,), jnp.int32),
            pltpu.VMEM((_B,), jnp.int32),
            pltpu.VMEM((_B,), jnp.int32),
            pltpu.VMEM((_B, 16), jnp.float32),
            pltpu.VMEM((_B, 16), jnp.float32),
            pltpu.VMEM((1024, 16), jnp.float32),
            pltpu.VMEM_SHARED((N, 16), jnp.float32),
            pltpu.SemaphoreType.DMA,
            pltpu.SemaphoreType.DMA,
        ],
    )
    def kern(z_h, src_h, dst_h, zeros_h, out0_h, out1_h, sb0, sb1, db0, db1,
             rb0, rb1, zb, acc, sg0, sg1):
        cid = lax.axis_index("c")
        sid = lax.axis_index("s")
        wid = cid * _NSUB + sid
        srcbs = (sb0, sb1)
        dstbs = (db0, db1)
        rowbs = (rb0, rb1)
        sems = (sg0, sg1)

        pltpu.sync_copy(zeros_h, zb)
        for j in range(6):
            pltpu.sync_copy(zb, acc.at[pl.ds(sid * _NT + j * 1024, 1024)])

        @pl.when(sid == 0)
        def _():
            pltpu.sync_copy(zb, acc.at[pl.ds(_NSUB * _NT, 1024)])
            pltpu.sync_copy(zb.at[pl.ds(0, _NEXTRA - 1024)],
                            acc.at[pl.ds(_NSUB * _NT + 1024, _NEXTRA - 1024)])

        plsc.subcore_barrier()

        def off(j):
            return (wid + j * _NW) * _B

        for b in range(2):
            pltpu.sync_copy(src_h.at[pl.ds(off(b), _B)], srcbs[b])
            pltpu.async_copy(z_h.at[srcbs[b]], rowbs[b], sems[b])
            pltpu.sync_copy(dst_h.at[pl.ds(off(b), _B)], dstbs[b])

        def step(it, carry):
            k = it * 2
            for b in range(2):
                j = k + b
                pltpu.make_async_copy(z_h.at[srcbs[b]], rowbs[b],
                                      sems[b]).wait()
                pltpu.sync_copy(rowbs[b], acc.at[dstbs[b]], add=True)

                @pl.when(j + 2 < _RPW)
                def _():
                    pltpu.sync_copy(src_h.at[pl.ds(off(j + 2), _B)], srcbs[b])
                    pltpu.async_copy(z_h.at[srcbs[b]], rowbs[b], sems[b])
                    pltpu.sync_copy(dst_h.at[pl.ds(off(j + 2), _B)], dstbs[b])

            return carry

        lax.fori_loop(0, _RPW // 2, step, 0)

        @pl.when(wid < _LEFT)
        def _():
            o = (_NW * _RPW + wid) * _B
            pltpu.sync_copy(src_h.at[pl.ds(o, _B)], sb0)
            pltpu.sync_copy(dst_h.at[pl.ds(o, _B)], db0)
            pltpu.async_copy(z_h.at[sb0], rb0, sg0).wait()
            pltpu.sync_copy(rb0, acc.at[db0], add=True)

        plsc.subcore_barrier()

        def copy_out(r0, nrows):
            pltpu.sync_copy(acc.at[pl.ds(r0, nrows)],
                            zb.at[pl.ds(0, nrows)])

            @pl.when(cid == 0)
            def _():
                pltpu.sync_copy(zb.at[pl.ds(0, nrows)],
                                out0_h.at[pl.ds(r0, nrows)])

            @pl.when(cid == 1)
            def _():
                pltpu.sync_copy(zb.at[pl.ds(0, nrows)],
                                out1_h.at[pl.ds(r0, nrows)])

        for j in range(6):
            copy_out(sid * _NT + j * 1024, 1024)

        @pl.when(sid == 0)
        def _():
            copy_out(_NSUB * _NT, 1024)
            copy_out(_NSUB * _NT + 1024, _NEXTRA - 1024)

    return kern(z, src1, dst1, zeros2d)


def _sc_preprocess(src1, dst1, cl1):
    @functools.partial(
        pl.kernel,
        out_type=[
            jax.ShapeDtypeStruct((N,), jnp.float32),
            jax.ShapeDtypeStruct((N,), jnp.float32),
            jax.ShapeDtypeStruct((_HISTP,), jnp.float32),
            jax.ShapeDtypeStruct((_HISTP,), jnp.float32),
        ],
        mesh=_SC_MESH,
        compiler_params=_SC_PARAMS,
        scratch_types=[
            pltpu.VMEM((_B,), jnp.int32),
            pltpu.VMEM((_B,), jnp.int32),
            pltpu.VMEM((_B,), jnp.int32),
            pltpu.VMEM((_B,), jnp.int32),
            pltpu.VMEM((_B,), jnp.int32),
            pltpu.VMEM((_B,), jnp.float32),
            pltpu.VMEM((2048,), jnp.float32),
            pltpu.VMEM_SHARED((N,), jnp.float32),
            pltpu.VMEM_SHARED((_HISTP,), jnp.float32),
            pltpu.SemaphoreType.DMA,
            pltpu.SemaphoreType.DMA,
        ],
    )
    def kern(src_h, dst_h, cl_h, deg0_h, deg1_h, h0_h, h1_h, sb, db, clsb,
             cldb, binb, ones, zb1, dega, hista, sg0, sg1):
        cid = lax.axis_index("c")
        sid = lax.axis_index("s")
        wid = cid * _NSUB + sid
        onev = jnp.ones((16,), jnp.float32)
        zv = jnp.zeros((16,), jnp.float32)
        for i in range(8):
            ones[pl.ds(i * 16, 16)] = onev

        def zrow(i, carry):
            zb1[pl.ds(i * 16, 16)] = zv
            return carry

        lax.fori_loop(0, 128, zrow, 0)
        for k in range(3):
            pltpu.sync_copy(zb1, dega.at[pl.ds(sid * _NT + k * 2048, 2048)])

        @pl.when(sid == 0)
        def _():
            pltpu.sync_copy(zb1.at[pl.ds(0, _NEXTRA)],
                            dega.at[pl.ds(_NSUB * _NT, _NEXTRA)])

        pltpu.sync_copy(zb1, hista.at[pl.ds(sid * _HT, 2048)])
        pltpu.sync_copy(zb1.at[pl.ds(0, _HT - 2048)],
                        hista.at[pl.ds(sid * _HT + 2048, _HT - 2048)])

        @pl.when(sid == 0)
        def _():
            pltpu.sync_copy(zb1.at[pl.ds(0, _HEXTRA)],
                            hista.at[pl.ds(_NSUB * _HT, _HEXTRA)])

        plsc.subcore_barrier()

        def row(o):
            pltpu.sync_copy(src_h.at[pl.ds(o, _B)], sb)
            pltpu.sync_copy(dst_h.at[pl.ds(o, _B)], db)
            pltpu.async_copy(cl_h.at[sb], clsb, sg0)
            pltpu.async_copy(cl_h.at[db], cldb, sg1)
            pltpu.make_async_copy(cl_h.at[sb], clsb, sg0).wait()
            pltpu.make_async_copy(cl_h.at[db], cldb, sg1).wait()
            for i in range(8):
                s = clsb[pl.ds(i * 16, 16)]
                dd = cldb[pl.ds(i * 16, 16)]
                binb[pl.ds(i * 16, 16)] = dd * HP + s
            pltpu.sync_copy(ones, hista.at[binb], add=True)
            pltpu.sync_copy(ones, dega.at[db], add=True)

        def step(k, carry):
            row((wid + k * _NW) * _B)
            return carry

        lax.fori_loop(0, _RPW, step, 0)

        @pl.when(wid < _LEFT)
        def _():
            row((_NW * _RPW + wid) * _B)

        plsc.subcore_barrier()

        def out_chunk(acc_ref, o0_h, o1_h, o, n):
            pltpu.sync_copy(acc_ref.at[pl.ds(o, n)], zb1.at[pl.ds(0, n)])

            @pl.when(cid == 0)
            def _():
                pltpu.sync_copy(zb1.at[pl.ds(0, n)], o0_h.at[pl.ds(o, n)])

            @pl.when(cid == 1)
            def _():
                pltpu.sync_copy(zb1.at[pl.ds(0, n)], o1_h.at[pl.ds(o, n)])

        for k in range(3):
            out_chunk(dega, deg0_h, deg1_h, sid * _NT + k * 2048, 2048)

        @pl.when(sid == 0)
        def _():
            out_chunk(dega, deg0_h, deg1_h, _NSUB * _NT, _NEXTRA)

        out_chunk(hista, h0_h, h1_h, sid * _HT, 2048)
        out_chunk(hista, h0_h, h1_h, sid * _HT + 2048, _HT - 2048)

        @pl.when(sid == 0)
        def _():
            out_chunk(hista, h0_h, h1_h, _NSUB * _HT, _HEXTRA)

    return kern(src1, dst1, cl1)


_PSTRIDE = 3072
_PCNT = 3360
_TBLP = 7296


def _sc_pool_max(x5f, cl1):
    @functools.partial(
        pl.kernel,
        out_type=jax.ShapeDtypeStruct((_NW * _TBLP,), jnp.float32),
        mesh=_SC_MESH,
        compiler_params=_SC_PARAMS,
        scratch_types=[
            pltpu.VMEM((_PCNT * 32,), jnp.float32),
            pltpu.VMEM((_PCNT,), jnp.int32),
            pltpu.VMEM((_TBLP,), jnp.float32),
        ],
    )
    def kern(x_h, cl_h, out_h, xb, clb, tbl):
        cid = lax.axis_index("c")
        sid = lax.axis_index("s")
        wid = cid * _NSUB + sid
        neg = jnp.full((16,), -jnp.inf, jnp.float32)

        def trow(i, carry):
            tbl[pl.ds(i * 16, 16)] = neg
            return carry

        lax.fori_loop(0, _TBLP // 16, trow, 0)

        base = jnp.where(wid == _NW - 1, N - _PCNT,
                         (wid * 3125) // 128 * 128)
        pltpu.sync_copy(x_h.at[pl.ds(base * 32, _PCNT * 32)], xb)
        pltpu.sync_copy(cl_h.at[pl.ds(base, _PCNT)], clb)

        def group(g, carry):
            cvec = clb[pl.ds(g * 16, 16)]
            for b in range(16):
                i = g * 16 + b
                cc = cvec[b]
                r0 = xb[pl.ds(i * 32, 16)]
                r1 = xb[pl.ds(i * 32 + 16, 16)]
                t0 = tbl[pl.ds(cc * 32, 16)]
                t1 = tbl[pl.ds(cc * 32 + 16, 16)]
                tbl[pl.ds(cc * 32, 16)] = jnp.maximum(t0, r0)
                tbl[pl.ds(cc * 32 + 16, 16)] = jnp.maximum(t1, r1)
            return carry

        lax.fori_loop(0, _PCNT // 16, group, 0)
        pltpu.sync_copy(tbl, out_h.at[pl.ds(wid * _TBLP, _TBLP)])

    return kern(x5f, cl1)


def kernel(x, pos, edge_index,
           W1, b1, gamma1, beta1,
           W2, b2, gamma2, beta2,
           W3, b3, gamma3, beta3,
           W4, b4, gamma4, beta4,
           W5, b5, gamma5, beta5,
           W6, b6, gamma6, beta6,
           W7, b7, gamma7, beta7,
           fcW):
    src1 = edge_index[0]
    dst1 = edge_index[1]
    zeros2d = jnp.zeros((1024, 16), jnp.float32)

    cl2 = _cl_from_pos(pos)
    cl = cl2.reshape(N)

    deg0, deg1, h0, h1 = _sc_preprocess(src1, dst1, cl)
    h0 = h0[:NC1 * HP].reshape(NC1, HP)
    h1 = h1[:NC1 * HP].reshape(NC1, HP)
    w1p = jnp.pad(W1, ((0, 0), (0, 8)))
    dinv, z = _tc1(deg0.reshape(N, 1), deg1.reshape(N, 1), x, w1p)

    stats16 = _make_stats(1)
    stats32 = _make_stats(2)
    epi_1 = _make_epilogue(16, 8, 16, residual=False, matmul=True)
    epi_mid = _make_epilogue(16, 16, 16, residual=False, matmul=True)
    epi_res = _make_epilogue(16, 16, 32, residual=True, matmul=True)
    epi_5 = _make_epilogue(32, 32, 0, residual=False, matmul=False)

    g = lambda a: a.reshape(1, -1)

    parts = _sc_aggregate(z, src1, dst1, zeros2d)
    p, st = stats16(parts, z, dinv)
    x1, z = epi_1(p, st, g(gamma1), g(beta1), W2, dinv)
    parts = _sc_aggregate(z, src1, dst1, zeros2d)
    p, st = stats16(parts, z, dinv)
    x2, z = epi_mid(p, st, g(gamma2), g(beta2), W3, dinv)
    parts = _sc_aggregate(z, src1, dst1, zeros2d)
    p, st = stats16(parts, z, dinv)
    x3, z = epi_mid(p, st, g(gamma3), g(beta3), W4, dinv)
    parts = _sc_aggregate(z, src1, dst1, zeros2d)
    p, st = stats16(parts, z, dinv)
    x4, z5 = epi_res(p, st, g(gamma4), g(beta4), W5, dinv, res=x2)
    parts_a = _sc_aggregate(z5[:, :16], src1, dst1, zeros2d)
    parts_b = _sc_aggregate(z5[:, 16:], src1, dst1, zeros2d)
    p, st = stats32(list(parts_a) + list(parts_b), z5, dinv)
    x5 = epi_5(p, st, g(gamma5), g(beta5))
    tflat = _sc_pool_max(x5.reshape(N * 32), cl)
    tables = tflat.reshape(_NW, _TBLP)[:, :NC1 * 32].reshape(_NW, NC1, 32)
    fcw3 = fcW.reshape(NC7, 32, 2)
    return _final(h0, h1, tables, W6, g(gamma6), g(beta6), W7, g(gamma7),
                  g(beta7), fcw3)


# speedup vs baseline: 42.2635x; 1.5101x over previous
import functools

import jax
import jax.numpy as jnp
from jax import lax
from jax.experimental import pallas as pl
from jax.experimental.pallas import tpu as pltpu
from jax.experimental.pallas import tpu_sc as plsc

N = 100000
E = 1600000
PX, PY = 16.0, 12.0
NX, NY = 15, 15
NC1 = NX * NY
HP = 232
P7X, P7Y = 60.0, 45.0
NC7 = 16
EPS = 1e-5

BR = 2000
GRID = N // BR


def _cl_body(pos_ref, cl_ref):
    p = pos_ref[...]
    cx = jnp.clip(jnp.floor(p[:, 0:1] / PX), 0, NX - 1)
    cy = jnp.clip(jnp.floor(p[:, 1:2] / PY), 0, NY - 1)
    cl_ref[...] = (cx * NY + cy).astype(jnp.int32)


def _cl_from_pos(pos):
    return pl.pallas_call(
        _cl_body,
        grid=(GRID,),
        in_specs=[pl.BlockSpec((BR, 2), lambda i: (i, 0))],
        out_specs=pl.BlockSpec((BR, 1), lambda i: (i, 0)),
        out_shape=jax.ShapeDtypeStruct((N, 1), jnp.int32),
    )(pos)


def _tc1_body(d0_ref, d1_ref, x_ref, w1_ref, dinv_ref, z1_ref):
    deg = d0_ref[...] + d1_ref[...] + 1.0
    dinv = lax.rsqrt(deg)
    dinv_ref[...] = dinv
    z1_ref[...] = (x_ref[...] * dinv) * w1_ref[...]


def _tc1(deg0, deg1, x, w1p):
    return pl.pallas_call(
        _tc1_body,
        grid=(GRID,),
        in_specs=[
            pl.BlockSpec((BR, 1), lambda i: (i, 0)),
            pl.BlockSpec((BR, 1), lambda i: (i, 0)),
            pl.BlockSpec((BR, 1), lambda i: (i, 0)),
            pl.BlockSpec((1, 16), lambda i: (0, 0)),
        ],
        out_specs=[
            pl.BlockSpec((BR, 1), lambda i: (i, 0)),
            pl.BlockSpec((BR, 16), lambda i: (i, 0)),
        ],
        out_shape=[
            jax.ShapeDtypeStruct((N, 1), jnp.float32),
            jax.ShapeDtypeStruct((N, 16), jnp.float32),
        ],
    )(deg0, deg1, x, w1p)


def _make_stats(nparts):
    C = 16 * nparts

    def body(*refs):
        part_refs = refs[:2 * nparts]
        z_ref, dinv_ref, p_ref, stats_ref = refs[2 * nparts:]
        agg = jnp.concatenate(
            [part_refs[2 * i][...] + part_refs[2 * i + 1][...]
             for i in range(nparts)], axis=1)
        p = dinv_ref[...] * (agg + z_ref[...])
        p_ref[...] = p
        s1 = jnp.sum(p, axis=0, keepdims=True)
        s2 = jnp.sum(p * p, axis=0, keepdims=True)
        upd = jnp.concatenate([s1, s2, jnp.zeros((6, C), jnp.float32)], axis=0)

        @pl.when(pl.program_id(0) == 0)
        def _():
            stats_ref[...] = jnp.zeros_like(stats_ref)

        stats_ref[...] += upd

    def call(parts, z, dinv):
        return pl.pallas_call(
            body,
            grid=(GRID,),
            in_specs=[pl.BlockSpec((BR, 16), lambda i: (i, 0))] * (2 * nparts)
            + [
                pl.BlockSpec((BR, C), lambda i: (i, 0)),
                pl.BlockSpec((BR, 1), lambda i: (i, 0)),
            ],
            out_specs=[
                pl.BlockSpec((BR, C), lambda i: (i, 0)),
                pl.BlockSpec((8, C), lambda i: (0, 0)),
            ],
            out_shape=[
                jax.ShapeDtypeStruct((N, C), jnp.float32),
                jax.ShapeDtypeStruct((8, C), jnp.float32),
            ],
        )(*parts, z, dinv)

    return call


def _make_epilogue(Cp, Cin, Cout, residual, matmul):
    def body(*refs):
        it = iter(refs)
        p_ref = next(it)
        stats_ref = next(it)
        g_ref = next(it)
        b_ref = next(it)
        w_ref = next(it) if matmul else None
        dinv_ref = next(it) if matmul else None
        res_ref = next(it) if residual else None
        x_ref = next(it)
        z_ref = next(it) if matmul else None

        p = p_ref[...][:, :Cin]
        mean = stats_ref[0:1, :Cin] / N
        var = stats_ref[1:2, :Cin] / N - mean * mean
        x = (p - mean) * lax.rsqrt(var + EPS) * g_ref[...] + b_ref[...]
        x = jnp.maximum(x, 0.0)
        if residual:
            x = x + res_ref[...]
        x_ref[...] = x
        if matmul:
            z_ref[...] = (
                jnp.dot(x, w_ref[...], preferred_element_type=jnp.float32)
                * dinv_ref[...]
            )

    def call(p, stats, gamma, beta, W=None, dinv=None, res=None):
        in_specs = [
            pl.BlockSpec((BR, Cp), lambda i: (i, 0)),
            pl.BlockSpec((8, Cp), lambda i: (0, 0)),
            pl.BlockSpec((1, Cin), lambda i: (0, 0)),
            pl.BlockSpec((1, Cin), lambda i: (0, 0)),
        ]
        args = [p, stats, gamma, beta]
        if matmul:
            in_specs.append(pl.BlockSpec((Cin, Cout), lambda i: (0, 0)))
            in_specs.append(pl.BlockSpec((BR, 1), lambda i: (i, 0)))
            args += [W, dinv]
        if residual:
            in_specs.append(pl.BlockSpec((BR, Cin), lambda i: (i, 0)))
            args.append(res)
        out_specs = [pl.BlockSpec((BR, Cin), lambda i: (i, 0))]
        out_shape = [jax.ShapeDtypeStruct((N, Cin), jnp.float32)]
        if matmul:
            out_specs.append(pl.BlockSpec((BR, Cout), lambda i: (i, 0)))
            out_shape.append(jax.ShapeDtypeStruct((N, Cout), jnp.float32))
        res_ = pl.pallas_call(
            body,
            grid=(GRID,),
            in_specs=in_specs,
            out_specs=out_specs,
            out_shape=out_shape,
        )(*args)
        return res_ if matmul else res_[0]

    return call


def _final_body(h0_ref, h1_ref, tbl_ref, w6_ref, g6_ref, b6_ref, w7_ref,
                g7_ref, b7_ref, fcw_ref, out_ref):
    cm = h0_ref[...] + h1_ref[...]
    cm = cm[:, :NC1]
    rows = lax.broadcasted_iota(jnp.int32, (NC1, NC1), 0)
    cols = lax.broadcasted_iota(jnp.int32, (NC1, NC1), 1)
    cm = cm + jnp.where(rows == cols, 1.0, 0.0)
    degp = jnp.sum(cm, axis=1, keepdims=True)
    dp = lax.rsqrt(degp)

    xp = jnp.max(tbl_ref[...], axis=0)
    xp = jnp.where(jnp.isfinite(xp), xp, 0.0)

    def bn_relu(p, g, b):
        mean = jnp.mean(p, axis=0, keepdims=True)
        var = jnp.mean(p * p, axis=0, keepdims=True) - mean * mean
        return jnp.maximum((p - mean) * lax.rsqrt(var + EPS) * g + b, 0.0)

    y = jnp.dot(xp, w6_ref[...], preferred_element_type=jnp.float32)
    p = dp * jnp.dot(cm, dp * y, preferred_element_type=jnp.float32)
    x = bn_relu(p, g6_ref[...], b6_ref[...])
    y = jnp.dot(x, w7_ref[...], preferred_element_type=jnp.float32)
    p = dp * jnp.dot(cm, dp * y, preferred_element_type=jnp.float32)
    x = bn_relu(p, g7_ref[...], b7_ref[...]) + xp

    ids = lax.broadcasted_iota(jnp.int32, (NC1, 1), 0)
    r = ids // NY
    c = ids % NY
    c7x = jnp.clip(jnp.floor((r.astype(jnp.float32) + 0.5) * PX / P7X), 0, 3)
    c7y = jnp.clip(jnp.floor((c.astype(jnp.float32) + 0.5) * PY / P7Y), 0, 3)
    c7 = (c7x * 4 + c7y).astype(jnp.int32)
    acc = jnp.zeros((1, 2), jnp.float32)
    for k in range(NC7):
        mk = jnp.max(jnp.where(c7 == k, x, -jnp.inf), axis=0, keepdims=True)
        mk = jnp.where(jnp.isfinite(mk), mk, 0.0)
        acc = acc + jnp.dot(mk, fcw_ref[k], preferred_element_type=jnp.float32)
    out_ref[...] = acc


def _final(h0, h1, tables, W6, g6, b6, W7, g7, b7, fcw3):
    return pl.pallas_call(
        _final_body,
        out_shape=jax.ShapeDtypeStruct((1, 2), jnp.float32),
    )(h0, h1, tables, W6, g6, b6, W7, g7, b7, fcw3)


_SC_MESH = plsc.VectorSubcoreMesh(core_axis_name="c", subcore_axis_name="s")
_SC_PARAMS = pltpu.CompilerParams(use_tc_tiling_on_sc=False)
_NCORE, _NSUB, _NW = 2, 16, 32
_B = 128
_ROWS = E // _B
_RPW = _ROWS // _NW
_LEFT = _ROWS - _NW * _RPW
_NT = 6144
_NEXTRA = N - _NSUB * _NT
_HISTP = 52224
_HT = 3200
_HEXTRA = _HISTP - _NSUB * _HT


_AB = 512
_AROWS = E // _AB
_AMAIN = 96
_ATAIL = _AROWS - _NW * _AMAIN
_ANB = 3
_AK = 2


def _sc_aggregate(z, src1, dst1, zeros2d):
    @functools.partial(
        pl.kernel,
        out_type=[
            jax.ShapeDtypeStruct((N, 16), jnp.float32),
            jax.ShapeDtypeStruct((N, 16), jnp.float32),
        ],
        mesh=_SC_MESH,
        compiler_params=_SC_PARAMS,
        scratch_types=[
            [pltpu.VMEM((_AB,), jnp.int32)] * _ANB,
            [pltpu.VMEM((_AB,), jnp.int32)] * _ANB,
            [pltpu.VMEM((_AB, 16), jnp.float32)] * _ANB,
            pltpu.VMEM_SHARED((N, 16), jnp.float32),
            [pltpu.SemaphoreType.DMA] * _ANB,
            [pltpu.SemaphoreType.DMA] * _ANB,
        ],
    )
    def kern(z_h, src_h, dst_h, zeros_h, out0_h, out1_h, sbs, dbs, rbs,
             acc, gsems, ssems):
        cid = lax.axis_index("c")
        sid = lax.axis_index("s")
        wid = cid * _NSUB + sid

        for j in range(6):
            pltpu.sync_copy(zeros_h, acc.at[pl.ds(sid * _NT + j * 1024, 1024)])

        @pl.when(sid == 0)
        def _():
            pltpu.sync_copy(zeros_h, acc.at[pl.ds(_NSUB * _NT, 1024)])
            pltpu.sync_copy(zeros_h.at[pl.ds(0, _NEXTRA - 1024)],
                            acc.at[pl.ds(_NSUB * _NT + 1024, _NEXTRA - 1024)])

        plsc.subcore_barrier()

        def off(j):
            return (wid + j * _NW) * _AB

        def fetch(j, b):
            pltpu.sync_copy(src_h.at[pl.ds(off(j), _AB)], sbs[b])
            pltpu.async_copy(z_h.at[sbs[b]], rbs[b], gsems[b])
            pltpu.sync_copy(dst_h.at[pl.ds(off(j), _AB)], dbs[b])

        for t in range(_AK):
            fetch(t, t)

        def step(it, carry):
            k = it * _ANB
            for b in range(_ANB):
                j = k + b
                pltpu.make_async_copy(z_h.at[sbs[b]], rbs[b],
                                      gsems[b]).wait()
                pltpu.async_copy(rbs[b], acc.at[dbs[b]], ssems[b], add=True)
                jp = j + _AK
                bp = (b + _AK) % _ANB

                @pl.when(jp < _AMAIN)
                def _():
                    @pl.when(jp >= _ANB)
                    def _():
                        pltpu.make_async_copy(rbs[bp], acc.at[dbs[bp]],
                                              ssems[bp]).wait()

                    fetch(jp, bp)

            return carry

        lax.fori_loop(0, _AMAIN // _ANB, step, 0)
        for b in range(_ANB):
            pltpu.make_async_copy(rbs[b], acc.at[dbs[b]], ssems[b]).wait()

        def tail_row(r):
            pltpu.sync_copy(src_h.at[pl.ds(r * _AB, _AB)], sbs[0])
            pltpu.sync_copy(dst_h.at[pl.ds(r * _AB, _AB)], dbs[0])
            pltpu.async_copy(z_h.at[sbs[0]], rbs[0], gsems[0]).wait()
            pltpu.sync_copy(rbs[0], acc.at[dbs[0]], add=True)

        tail_row(_NW * _AMAIN + wid)

        @pl.when(wid < _ATAIL - _NW)
        def _():
            tail_row(_NW * _AMAIN + _NW + wid)

        plsc.subcore_barrier()

        def copy_out(r0, nrows):
            @pl.when(cid == 0)
            def _():
                pltpu.sync_copy(acc.at[pl.ds(r0, nrows)],
                                out0_h.at[pl.ds(r0, nrows)])

            @pl.when(cid == 1)
            def _():
                pltpu.sync_copy(acc.at[pl.ds(r0, nrows)],
                                out1_h.at[pl.ds(r0, nrows)])

        for j in range(3):
            copy_out(sid * _NT + j * 2048, 2048)

        @pl.when(sid == 0)
        def _():
            copy_out(_NSUB * _NT, _NEXTRA)

    return kern(z, src1, dst1, zeros2d)


def _sc_preprocess(src1, dst1, cl1):
    @functools.partial(
        pl.kernel,
        out_type=[
            jax.ShapeDtypeStruct((N,), jnp.float32),
            jax.ShapeDtypeStruct((N,), jnp.float32),
            jax.ShapeDtypeStruct((_HISTP,), jnp.float32),
            jax.ShapeDtypeStruct((_HISTP,), jnp.float32),
        ],
        mesh=_SC_MESH,
        compiler_params=_SC_PARAMS,
        scratch_types=[
            pltpu.VMEM((_B,), jnp.int32),
            pltpu.VMEM((_B,), jnp.int32),
            pltpu.VMEM((_B,), jnp.int32),
            pltpu.VMEM((_B,), jnp.int32),
            pltpu.VMEM((_B,), jnp.int32),
            pltpu.VMEM((_B,), jnp.float32),
            pltpu.VMEM((2048,), jnp.float32),
            pltpu.VMEM_SHARED((N,), jnp.float32),
            pltpu.VMEM_SHARED((_HISTP,), jnp.float32),
            pltpu.SemaphoreType.DMA,
            pltpu.SemaphoreType.DMA,
        ],
    )
    def kern(src_h, dst_h, cl_h, deg0_h, deg1_h, h0_h, h1_h, sb, db, clsb,
             cldb, binb, ones, zb1, dega, hista, sg0, sg1):
        cid = lax.axis_index("c")
        sid = lax.axis_index("s")
        wid = cid * _NSUB + sid
        onev = jnp.ones((16,), jnp.float32)
        zv = jnp.zeros((16,), jnp.float32)
        for i in range(8):
            ones[pl.ds(i * 16, 16)] = onev

        def zrow(i, carry):
            zb1[pl.ds(i * 16, 16)] = zv
            return carry

        lax.fori_loop(0, 128, zrow, 0)
        for k in range(3):
            pltpu.sync_copy(zb1, dega.at[pl.ds(sid * _NT + k * 2048, 2048)])

        @pl.when(sid == 0)
        def _():
            pltpu.sync_copy(zb1.at[pl.ds(0, _NEXTRA)],
                            dega.at[pl.ds(_NSUB * _NT, _NEXTRA)])

        pltpu.sync_copy(zb1, hista.at[pl.ds(sid * _HT, 2048)])
        pltpu.sync_copy(zb1.at[pl.ds(0, _HT - 2048)],
                        hista.at[pl.ds(sid * _HT + 2048, _HT - 2048)])

        @pl.when(sid == 0)
        def _():
            pltpu.sync_copy(zb1.at[pl.ds(0, _HEXTRA)],
                            hista.at[pl.ds(_NSUB * _HT, _HEXTRA)])

        plsc.subcore_barrier()

        def row(o):
            pltpu.sync_copy(src_h.at[pl.ds(o, _B)], sb)
            pltpu.sync_copy(dst_h.at[pl.ds(o, _B)], db)
            pltpu.async_copy(cl_h.at[sb], clsb, sg0)
            pltpu.async_copy(cl_h.at[db], cldb, sg1)
            pltpu.make_async_copy(cl_h.at[sb], clsb, sg0).wait()
            pltpu.make_async_copy(cl_h.at[db], cldb, sg1).wait()
            for i in range(8):
                s = clsb[pl.ds(i * 16, 16)]
                dd = cldb[pl.ds(i * 16, 16)]
                binb[pl.ds(i * 16, 16)] = dd * HP + s
            pltpu.sync_copy(ones, hista.at[binb], add=True)
            pltpu.sync_copy(ones, dega.at[db], add=True)

        def step(k, carry):
            row((wid + k * _NW) * _B)
            return carry

        lax.fori_loop(0, _RPW, step, 0)

        @pl.when(wid < _LEFT)
        def _():
            row((_NW * _RPW + wid) * _B)

        plsc.subcore_barrier()

        def out_chunk(acc_ref, o0_h, o1_h, o, n):
            pltpu.sync_copy(acc_ref.at[pl.ds(o, n)], zb1.at[pl.ds(0, n)])

            @pl.when(cid == 0)
            def _():
                pltpu.sync_copy(zb1.at[pl.ds(0, n)], o0_h.at[pl.ds(o, n)])

            @pl.when(cid == 1)
            def _():
                pltpu.sync_copy(zb1.at[pl.ds(0, n)], o1_h.at[pl.ds(o, n)])

        for k in range(3):
            out_chunk(dega, deg0_h, deg1_h, sid * _NT + k * 2048, 2048)

        @pl.when(sid == 0)
        def _():
            out_chunk(dega, deg0_h, deg1_h, _NSUB * _NT, _NEXTRA)

        out_chunk(hista, h0_h, h1_h, sid * _HT, 2048)
        out_chunk(hista, h0_h, h1_h, sid * _HT + 2048, _HT - 2048)

        @pl.when(sid == 0)
        def _():
            out_chunk(hista, h0_h, h1_h, _NSUB * _HT, _HEXTRA)

    return kern(src1, dst1, cl1)


_PSTRIDE = 3072
_PCNT = 3360
_TBLP = 7296


def _sc_pool_max(x5f, cl1):
    @functools.partial(
        pl.kernel,
        out_type=jax.ShapeDtypeStruct((_NW * _TBLP,), jnp.float32),
        mesh=_SC_MESH,
        compiler_params=_SC_PARAMS,
        scratch_types=[
            pltpu.VMEM((_PCNT * 32,), jnp.float32),
            pltpu.VMEM((_PCNT,), jnp.int32),
            pltpu.VMEM((_TBLP,), jnp.float32),
        ],
    )
    def kern(x_h, cl_h, out_h, xb, clb, tbl):
        cid = lax.axis_index("c")
        sid = lax.axis_index("s")
        wid = cid * _NSUB + sid
        neg = jnp.full((16,), -jnp.inf, jnp.float32)

        def trow(i, carry):
            tbl[pl.ds(i * 16, 16)] = neg
            return carry

        lax.fori_loop(0, _TBLP // 16, trow, 0)

        base = jnp.where(wid == _NW - 1, N - _PCNT,
                         (wid * 3125) // 128 * 128)
        pltpu.sync_copy(x_h.at[pl.ds(base * 32, _PCNT * 32)], xb)
        pltpu.sync_copy(cl_h.at[pl.ds(base, _PCNT)], clb)

        def group(g, carry):
            cvec = clb[pl.ds(g * 16, 16)]
            for b in range(16):
                i = g * 16 + b
                cc = cvec[b]
                r0 = xb[pl.ds(i * 32, 16)]
                r1 = xb[pl.ds(i * 32 + 16, 16)]
                t0 = tbl[pl.ds(cc * 32, 16)]
                t1 = tbl[pl.ds(cc * 32 + 16, 16)]
                tbl[pl.ds(cc * 32, 16)] = jnp.maximum(t0, r0)
                tbl[pl.ds(cc * 32 + 16, 16)] = jnp.maximum(t1, r1)
            return carry

        lax.fori_loop(0, _PCNT // 16, group, 0)
        pltpu.sync_copy(tbl, out_h.at[pl.ds(wid * _TBLP, _TBLP)])

    return kern(x5f, cl1)


def kernel(x, pos, edge_index,
           W1, b1, gamma1, beta1,
           W2, b2, gamma2, beta2,
           W3, b3, gamma3, beta3,
           W4, b4, gamma4, beta4,
           W5, b5, gamma5, beta5,
           W6, b6, gamma6, beta6,
           W7, b7, gamma7, beta7,
           fcW):
    src1 = edge_index[0]
    dst1 = edge_index[1]
    zeros2d = jnp.zeros((1024, 16), jnp.float32)

    cl2 = _cl_from_pos(pos)
    cl = cl2.reshape(N)

    deg0, deg1, h0, h1 = _sc_preprocess(src1, dst1, cl)
    h0 = h0[:NC1 * HP].reshape(NC1, HP)
    h1 = h1[:NC1 * HP].reshape(NC1, HP)
    w1p = jnp.pad(W1, ((0, 0), (0, 8)))
    dinv, z = _tc1(deg0.reshape(N, 1), deg1.reshape(N, 1), x, w1p)

    stats16 = _make_stats(1)
    stats32 = _make_stats(2)
    epi_1 = _make_epilogue(16, 8, 16, residual=False, matmul=True)
    epi_mid = _make_epilogue(16, 16, 16, residual=False, matmul=True)
    epi_res = _make_epilogue(16, 16, 32, residual=True, matmul=True)
    epi_5 = _make_epilogue(32, 32, 0, residual=False, matmul=False)

    g = lambda a: a.reshape(1, -1)

    parts = _sc_aggregate(z, src1, dst1, zeros2d)
    p, st = stats16(parts, z, dinv)
    x1, z = epi_1(p, st, g(gamma1), g(beta1), W2, dinv)
    parts = _sc_aggregate(z, src1, dst1, zeros2d)
    p, st = stats16(parts, z, dinv)
    x2, z = epi_mid(p, st, g(gamma2), g(beta2), W3, dinv)
    parts = _sc_aggregate(z, src1, dst1, zeros2d)
    p, st = stats16(parts, z, dinv)
    x3, z = epi_mid(p, st, g(gamma3), g(beta3), W4, dinv)
    parts = _sc_aggregate(z, src1, dst1, zeros2d)
    p, st = stats16(parts, z, dinv)
    x4, z5 = epi_res(p, st, g(gamma4), g(beta4), W5, dinv, res=x2)
    parts_a = _sc_aggregate(z5[:, :16], src1, dst1, zeros2d)
    parts_b = _sc_aggregate(z5[:, 16:], src1, dst1, zeros2d)
    p, st = stats32(list(parts_a) + list(parts_b), z5, dinv)
    x5 = epi_5(p, st, g(gamma5), g(beta5))
    tflat = _sc_pool_max(x5.reshape(N * 32), cl)
    tables = tflat.reshape(_NW, _TBLP)[:, :NC1 * 32].reshape(_NW, NC1, 32)
    fcw3 = fcW.reshape(NC7, 32, 2)
    return _final(h0, h1, tables, W6, g(gamma6), g(beta6), W7, g(gamma7),
                  g(beta7), fcw3)


# speedup vs baseline: 49.6517x; 1.1748x over previous
import functools

import jax
import jax.numpy as jnp
from jax import lax
from jax.experimental import pallas as pl
from jax.experimental.pallas import tpu as pltpu
from jax.experimental.pallas import tpu_sc as plsc

N = 100000
E = 1600000
PX, PY = 16.0, 12.0
NX, NY = 15, 15
NC1 = NX * NY
HP = 232
P7X, P7Y = 60.0, 45.0
NC7 = 16
EPS = 1e-5

BR = 2000
GRID = N // BR


def _cl_body(pos_ref, cl_ref):
    p = pos_ref[...]
    cx = jnp.clip(jnp.floor(p[:, 0:1] / PX), 0, NX - 1)
    cy = jnp.clip(jnp.floor(p[:, 1:2] / PY), 0, NY - 1)
    cl_ref[...] = (cx * NY + cy).astype(jnp.int32)


def _cl_from_pos(pos):
    return pl.pallas_call(
        _cl_body,
        grid=(GRID,),
        in_specs=[pl.BlockSpec((BR, 2), lambda i: (i, 0))],
        out_specs=pl.BlockSpec((BR, 1), lambda i: (i, 0)),
        out_shape=jax.ShapeDtypeStruct((N, 1), jnp.int32),
    )(pos)


def _tc1_body(d0_ref, d1_ref, x_ref, w1_ref, dinv_ref, z1_ref):
    deg = d0_ref[...] + d1_ref[...] + 1.0
    dinv = lax.rsqrt(deg)
    dinv_ref[...] = dinv
    z1_ref[...] = (x_ref[...] * dinv) * w1_ref[...]


def _tc1(deg0, deg1, x, w1p):
    return pl.pallas_call(
        _tc1_body,
        grid=(GRID,),
        in_specs=[
            pl.BlockSpec((BR, 1), lambda i: (i, 0)),
            pl.BlockSpec((BR, 1), lambda i: (i, 0)),
            pl.BlockSpec((BR, 1), lambda i: (i, 0)),
            pl.BlockSpec((1, 16), lambda i: (0, 0)),
        ],
        out_specs=[
            pl.BlockSpec((BR, 1), lambda i: (i, 0)),
            pl.BlockSpec((BR, 16), lambda i: (i, 0)),
        ],
        out_shape=[
            jax.ShapeDtypeStruct((N, 1), jnp.float32),
            jax.ShapeDtypeStruct((N, 16), jnp.float32),
        ],
    )(deg0, deg1, x, w1p)


def _make_stats(nparts):
    C = 16 * nparts

    def body(*refs):
        part_refs = refs[:2 * nparts]
        z_ref, dinv_ref, p_ref, stats_ref = refs[2 * nparts:]
        agg = jnp.concatenate(
            [part_refs[2 * i][...] + part_refs[2 * i + 1][...]
             for i in range(nparts)], axis=1)
        p = dinv_ref[...] * (agg + z_ref[...])
        p_ref[...] = p
        s1 = jnp.sum(p, axis=0, keepdims=True)
        s2 = jnp.sum(p * p, axis=0, keepdims=True)
        upd = jnp.concatenate([s1, s2, jnp.zeros((6, C), jnp.float32)], axis=0)

        @pl.when(pl.program_id(0) == 0)
        def _():
            stats_ref[...] = jnp.zeros_like(stats_ref)

        stats_ref[...] += upd

    def call(parts, z, dinv):
        return pl.pallas_call(
            body,
            grid=(GRID,),
            in_specs=[pl.BlockSpec((BR, 16), lambda i: (i, 0))] * (2 * nparts)
            + [
                pl.BlockSpec((BR, C), lambda i: (i, 0)),
                pl.BlockSpec((BR, 1), lambda i: (i, 0)),
            ],
            out_specs=[
                pl.BlockSpec((BR, C), lambda i: (i, 0)),
                pl.BlockSpec((8, C), lambda i: (0, 0)),
            ],
            out_shape=[
                jax.ShapeDtypeStruct((N, C), jnp.float32),
                jax.ShapeDtypeStruct((8, C), jnp.float32),
            ],
        )(*parts, z, dinv)

    return call


def _make_epilogue(Cp, Cin, Cout, residual, matmul):
    def body(*refs):
        it = iter(refs)
        p_ref = next(it)
        stats_ref = next(it)
        g_ref = next(it)
        b_ref = next(it)
        w_ref = next(it) if matmul else None
        dinv_ref = next(it) if matmul else None
        res_ref = next(it) if residual else None
        x_ref = next(it)
        z_ref = next(it) if matmul else None

        p = p_ref[...][:, :Cin]
        mean = stats_ref[0:1, :Cin] / N
        var = stats_ref[1:2, :Cin] / N - mean * mean
        x = (p - mean) * lax.rsqrt(var + EPS) * g_ref[...] + b_ref[...]
        x = jnp.maximum(x, 0.0)
        if residual:
            x = x + res_ref[...]
        x_ref[...] = x
        if matmul:
            z_ref[...] = (
                jnp.dot(x, w_ref[...], preferred_element_type=jnp.float32)
                * dinv_ref[...]
            )

    def call(p, stats, gamma, beta, W=None, dinv=None, res=None):
        in_specs = [
            pl.BlockSpec((BR, Cp), lambda i: (i, 0)),
            pl.BlockSpec((8, Cp), lambda i: (0, 0)),
            pl.BlockSpec((1, Cin), lambda i: (0, 0)),
            pl.BlockSpec((1, Cin), lambda i: (0, 0)),
        ]
        args = [p, stats, gamma, beta]
        if matmul:
            in_specs.append(pl.BlockSpec((Cin, Cout), lambda i: (0, 0)))
            in_specs.append(pl.BlockSpec((BR, 1), lambda i: (i, 0)))
            args += [W, dinv]
        if residual:
            in_specs.append(pl.BlockSpec((BR, Cin), lambda i: (i, 0)))
            args.append(res)
        out_specs = [pl.BlockSpec((BR, Cin), lambda i: (i, 0))]
        out_shape = [jax.ShapeDtypeStruct((N, Cin), jnp.float32)]
        if matmul:
            out_specs.append(pl.BlockSpec((BR, Cout), lambda i: (i, 0)))
            out_shape.append(jax.ShapeDtypeStruct((N, Cout), jnp.float32))
        res_ = pl.pallas_call(
            body,
            grid=(GRID,),
            in_specs=in_specs,
            out_specs=out_specs,
            out_shape=out_shape,
        )(*args)
        return res_ if matmul else res_[0]

    return call


def _final_body(h0_ref, h1_ref, tbl_ref, w6_ref, g6_ref, b6_ref, w7_ref,
                g7_ref, b7_ref, fcw_ref, out_ref):
    cm = h0_ref[...] + h1_ref[...]
    cm = cm[:, :NC1]
    rows = lax.broadcasted_iota(jnp.int32, (NC1, NC1), 0)
    cols = lax.broadcasted_iota(jnp.int32, (NC1, NC1), 1)
    cm = cm + jnp.where(rows == cols, 1.0, 0.0)
    degp = jnp.sum(cm, axis=1, keepdims=True)
    dp = lax.rsqrt(degp)

    xp = jnp.max(tbl_ref[...], axis=0)
    xp = jnp.where(jnp.isfinite(xp), xp, 0.0)

    def bn_relu(p, g, b):
        mean = jnp.mean(p, axis=0, keepdims=True)
        var = jnp.mean(p * p, axis=0, keepdims=True) - mean * mean
        return jnp.maximum((p - mean) * lax.rsqrt(var + EPS) * g + b, 0.0)

    y = jnp.dot(xp, w6_ref[...], preferred_element_type=jnp.float32)
    p = dp * jnp.dot(cm, dp * y, preferred_element_type=jnp.float32)
    x = bn_relu(p, g6_ref[...], b6_ref[...])
    y = jnp.dot(x, w7_ref[...], preferred_element_type=jnp.float32)
    p = dp * jnp.dot(cm, dp * y, preferred_element_type=jnp.float32)
    x = bn_relu(p, g7_ref[...], b7_ref[...]) + xp

    ids = lax.broadcasted_iota(jnp.int32, (NC1, 1), 0)
    r = ids // NY
    c = ids % NY
    c7x = jnp.clip(jnp.floor((r.astype(jnp.float32) + 0.5) * PX / P7X), 0, 3)
    c7y = jnp.clip(jnp.floor((c.astype(jnp.float32) + 0.5) * PY / P7Y), 0, 3)
    c7 = (c7x * 4 + c7y).astype(jnp.int32)
    acc = jnp.zeros((1, 2), jnp.float32)
    for k in range(NC7):
        mk = jnp.max(jnp.where(c7 == k, x, -jnp.inf), axis=0, keepdims=True)
        mk = jnp.where(jnp.isfinite(mk), mk, 0.0)
        acc = acc + jnp.dot(mk, fcw_ref[k], preferred_element_type=jnp.float32)
    out_ref[...] = acc


def _final(h0, h1, tables, W6, g6, b6, W7, g7, b7, fcw3):
    return pl.pallas_call(
        _final_body,
        out_shape=jax.ShapeDtypeStruct((1, 2), jnp.float32),
    )(h0, h1, tables, W6, g6, b6, W7, g7, b7, fcw3)


_SC_MESH = plsc.VectorSubcoreMesh(core_axis_name="c", subcore_axis_name="s")
_SC_PARAMS = pltpu.CompilerParams(use_tc_tiling_on_sc=False)
_NCORE, _NSUB, _NW = 2, 16, 32
_B = 128
_ROWS = E // _B
_RPW = _ROWS // _NW
_LEFT = _ROWS - _NW * _RPW
_NT = 6144
_NEXTRA = N - _NSUB * _NT
_HISTP = 52224
_HT = 3200
_HEXTRA = _HISTP - _NSUB * _HT


_AB = 512
_AROWS = E // _AB
_AMAIN = 96
_ATAIL = _AROWS - _NW * _AMAIN
_ANB = 3
_AK = 2


def _sc_aggregate(z, src1, dst1, zeros2d):
    @functools.partial(
        pl.kernel,
        out_type=[
            jax.ShapeDtypeStruct((N, 16), jnp.float32),
            jax.ShapeDtypeStruct((N, 16), jnp.float32),
        ],
        mesh=_SC_MESH,
        compiler_params=_SC_PARAMS,
        scratch_types=[
            [pltpu.VMEM((_AB,), jnp.int32)] * _ANB,
            [pltpu.VMEM((_AB,), jnp.int32)] * _ANB,
            [pltpu.VMEM((_AB, 16), jnp.float32)] * _ANB,
            pltpu.VMEM_SHARED((N, 16), jnp.float32),
            [pltpu.SemaphoreType.DMA] * _ANB,
            [pltpu.SemaphoreType.DMA] * _ANB,
        ],
    )
    def kern(z_h, src_h, dst_h, zeros_h, out0_h, out1_h, sbs, dbs, rbs,
             acc, gsems, ssems):
        cid = lax.axis_index("c")
        sid = lax.axis_index("s")
        wid = cid * _NSUB + sid

        for j in range(6):
            pltpu.sync_copy(zeros_h, acc.at[pl.ds(sid * _NT + j * 1024, 1024)])

        @pl.when(sid == 0)
        def _():
            pltpu.sync_copy(zeros_h, acc.at[pl.ds(_NSUB * _NT, 1024)])
            pltpu.sync_copy(zeros_h.at[pl.ds(0, _NEXTRA - 1024)],
                            acc.at[pl.ds(_NSUB * _NT + 1024, _NEXTRA - 1024)])

        plsc.subcore_barrier()

        def off(j):
            return (wid + j * _NW) * _AB

        def fetch(j, b):
            pltpu.sync_copy(src_h.at[pl.ds(off(j), _AB)], sbs[b])
            pltpu.async_copy(z_h.at[sbs[b]], rbs[b], gsems[b])
            pltpu.sync_copy(dst_h.at[pl.ds(off(j), _AB)], dbs[b])

        for t in range(_AK):
            fetch(t, t)

        def step(it, carry):
            k = it * _ANB
            for b in range(_ANB):
                j = k + b
                pltpu.make_async_copy(z_h.at[sbs[b]], rbs[b],
                                      gsems[b]).wait()
                pltpu.async_copy(rbs[b], acc.at[dbs[b]], ssems[b], add=True)
                jp = j + _AK
                bp = (b + _AK) % _ANB

                @pl.when(jp < _AMAIN)
                def _():
                    @pl.when(jp >= _ANB)
                    def _():
                        pltpu.make_async_copy(rbs[bp], acc.at[dbs[bp]],
                                              ssems[bp]).wait()

                    fetch(jp, bp)

            return carry

        lax.fori_loop(0, _AMAIN // _ANB, step, 0)
        for b in range(_ANB):
            pltpu.make_async_copy(rbs[b], acc.at[dbs[b]], ssems[b]).wait()

        def tail_row(r):
            pltpu.sync_copy(src_h.at[pl.ds(r * _AB, _AB)], sbs[0])
            pltpu.sync_copy(dst_h.at[pl.ds(r * _AB, _AB)], dbs[0])
            pltpu.async_copy(z_h.at[sbs[0]], rbs[0], gsems[0]).wait()
            pltpu.sync_copy(rbs[0], acc.at[dbs[0]], add=True)

        tail_row(_NW * _AMAIN + wid)

        @pl.when(wid < _ATAIL - _NW)
        def _():
            tail_row(_NW * _AMAIN + _NW + wid)

        plsc.subcore_barrier()

        def copy_out(r0, nrows):
            @pl.when(cid == 0)
            def _():
                pltpu.sync_copy(acc.at[pl.ds(r0, nrows)],
                                out0_h.at[pl.ds(r0, nrows)])

            @pl.when(cid == 1)
            def _():
                pltpu.sync_copy(acc.at[pl.ds(r0, nrows)],
                                out1_h.at[pl.ds(r0, nrows)])

        for j in range(3):
            copy_out(sid * _NT + j * 2048, 2048)

        @pl.when(sid == 0)
        def _():
            copy_out(_NSUB * _NT, _NEXTRA)

    return kern(z, src1, dst1, zeros2d)


def _sc_preprocess(src1, dst1, cl1, zeros1d):
    @functools.partial(
        pl.kernel,
        out_type=[
            jax.ShapeDtypeStruct((N,), jnp.float32),
            jax.ShapeDtypeStruct((N,), jnp.float32),
            jax.ShapeDtypeStruct((_HISTP,), jnp.float32),
            jax.ShapeDtypeStruct((_HISTP,), jnp.float32),
        ],
        mesh=_SC_MESH,
        compiler_params=_SC_PARAMS,
        scratch_types=[
            [pltpu.VMEM((_AB,), jnp.int32)] * _ANB,
            [pltpu.VMEM((_AB,), jnp.int32)] * _ANB,
            [pltpu.VMEM((_AB,), jnp.int32)] * _ANB,
            [pltpu.VMEM((_AB,), jnp.int32)] * _ANB,
            pltpu.VMEM((_AB,), jnp.int32),
            pltpu.VMEM((_AB,), jnp.float32),
            pltpu.VMEM_SHARED((N,), jnp.float32),
            pltpu.VMEM_SHARED((_HISTP,), jnp.float32),
            [pltpu.SemaphoreType.DMA] * _ANB,
            [pltpu.SemaphoreType.DMA] * _ANB,
        ],
    )
    def kern(src_h, dst_h, cl_h, zeros_h, deg0_h, deg1_h, h0_h, h1_h, sbs,
             dbs, clsbs, cldbs, binb, ones, dega, hista, gss, gds):
        cid = lax.axis_index("c")
        sid = lax.axis_index("s")
        wid = cid * _NSUB + sid
        onev = jnp.ones((16,), jnp.float32)
        for i in range(_AB // 16):
            ones[pl.ds(i * 16, 16)] = onev

        for k in range(3):
            pltpu.sync_copy(zeros_h,
                            dega.at[pl.ds(sid * _NT + k * 2048, 2048)])

        @pl.when(sid == 0)
        def _():
            pltpu.sync_copy(zeros_h.at[pl.ds(0, _NEXTRA)],
                            dega.at[pl.ds(_NSUB * _NT, _NEXTRA)])

        pltpu.sync_copy(zeros_h, hista.at[pl.ds(sid * _HT, 2048)])
        pltpu.sync_copy(zeros_h.at[pl.ds(0, _HT - 2048)],
                        hista.at[pl.ds(sid * _HT + 2048, _HT - 2048)])

        @pl.when(sid == 0)
        def _():
            pltpu.sync_copy(zeros_h.at[pl.ds(0, _HEXTRA)],
                            hista.at[pl.ds(_NSUB * _HT, _HEXTRA)])

        plsc.subcore_barrier()

        def off(j):
            return (wid + j * _NW) * _AB

        def fetch(j, b):
            pltpu.sync_copy(src_h.at[pl.ds(off(j), _AB)], sbs[b])
            pltpu.async_copy(cl_h.at[sbs[b]], clsbs[b], gss[b])
            pltpu.sync_copy(dst_h.at[pl.ds(off(j), _AB)], dbs[b])
            pltpu.async_copy(cl_h.at[dbs[b]], cldbs[b], gds[b])

        for t in range(_AK):
            fetch(t, t)

        def step(it, carry):
            k = it * _ANB
            for b in range(_ANB):
                j = k + b
                pltpu.make_async_copy(cl_h.at[sbs[b]], clsbs[b],
                                      gss[b]).wait()
                pltpu.make_async_copy(cl_h.at[dbs[b]], cldbs[b],
                                      gds[b]).wait()
                for i in range(_AB // 16):
                    s = clsbs[b][pl.ds(i * 16, 16)]
                    dd = cldbs[b][pl.ds(i * 16, 16)]
                    binb[pl.ds(i * 16, 16)] = dd * HP + s
                pltpu.sync_copy(ones, hista.at[binb], add=True)
                pltpu.sync_copy(ones, dega.at[dbs[b]], add=True)
                jp = j + _AK
                bp = (b + _AK) % _ANB

                @pl.when(jp < _AMAIN)
                def _():
                    fetch(jp, bp)

            return carry

        lax.fori_loop(0, _AMAIN // _ANB, step, 0)

        def tail_row(r):
            o = r * _AB
            pltpu.sync_copy(src_h.at[pl.ds(o, _AB)], sbs[0])
            pltpu.sync_copy(dst_h.at[pl.ds(o, _AB)], dbs[0])
            pltpu.async_copy(cl_h.at[sbs[0]], clsbs[0], gss[0])
            pltpu.async_copy(cl_h.at[dbs[0]], cldbs[0], gds[0])
            pltpu.make_async_copy(cl_h.at[sbs[0]], clsbs[0], gss[0]).wait()
            pltpu.make_async_copy(cl_h.at[dbs[0]], cldbs[0], gds[0]).wait()
            for i in range(_AB // 16):
                s = clsbs[0][pl.ds(i * 16, 16)]
                dd = cldbs[0][pl.ds(i * 16, 16)]
                binb[pl.ds(i * 16, 16)] = dd * HP + s
            pltpu.sync_copy(ones, hista.at[binb], add=True)
            pltpu.sync_copy(ones, dega.at[dbs[0]], add=True)

        tail_row(_NW * _AMAIN + wid)

        @pl.when(wid < _ATAIL - _NW)
        def _():
            tail_row(_NW * _AMAIN + _NW + wid)

        plsc.subcore_barrier()

        def out_chunk(acc_ref, o0_h, o1_h, o, n):
            @pl.when(cid == 0)
            def _():
                pltpu.sync_copy(acc_ref.at[pl.ds(o, n)],
                                o0_h.at[pl.ds(o, n)])

            @pl.when(cid == 1)
            def _():
                pltpu.sync_copy(acc_ref.at[pl.ds(o, n)],
                                o1_h.at[pl.ds(o, n)])

        for k in range(3):
            out_chunk(dega, deg0_h, deg1_h, sid * _NT + k * 2048, 2048)

        @pl.when(sid == 0)
        def _():
            out_chunk(dega, deg0_h, deg1_h, _NSUB * _NT, _NEXTRA)

        out_chunk(hista, h0_h, h1_h, sid * _HT, 2048)
        out_chunk(hista, h0_h, h1_h, sid * _HT + 2048, _HT - 2048)

        @pl.when(sid == 0)
        def _():
            out_chunk(hista, h0_h, h1_h, _NSUB * _HT, _HEXTRA)

    return kern(src1, dst1, cl1, zeros1d)


_PSTRIDE = 3072
_PCNT = 3360
_TBLP = 7296


def _sc_pool_max(x5f, cl1):
    @functools.partial(
        pl.kernel,
        out_type=jax.ShapeDtypeStruct((_NW * _TBLP,), jnp.float32),
        mesh=_SC_MESH,
        compiler_params=_SC_PARAMS,
        scratch_types=[
            pltpu.VMEM((_PCNT * 32,), jnp.float32),
            pltpu.VMEM((_PCNT,), jnp.int32),
            pltpu.VMEM((_TBLP,), jnp.float32),
        ],
    )
    def kern(x_h, cl_h, out_h, xb, clb, tbl):
        cid = lax.axis_index("c")
        sid = lax.axis_index("s")
        wid = cid * _NSUB + sid
        neg = jnp.full((16,), -jnp.inf, jnp.float32)

        def trow(i, carry):
            tbl[pl.ds(i * 16, 16)] = neg
            return carry

        lax.fori_loop(0, _TBLP // 16, trow, 0)

        base = jnp.where(wid == _NW - 1, N - _PCNT,
                         (wid * 3125) // 128 * 128)
        pltpu.sync_copy(x_h.at[pl.ds(base * 32, _PCNT * 32)], xb)
        pltpu.sync_copy(cl_h.at[pl.ds(base, _PCNT)], clb)

        def group(g, carry):
            cvec = clb[pl.ds(g * 16, 16)]
            for b in range(16):
                i = g * 16 + b
                cc = cvec[b]
                r0 = xb[pl.ds(i * 32, 16)]
                r1 = xb[pl.ds(i * 32 + 16, 16)]
                t0 = tbl[pl.ds(cc * 32, 16)]
                t1 = tbl[pl.ds(cc * 32 + 16, 16)]
                tbl[pl.ds(cc * 32, 16)] = jnp.maximum(t0, r0)
                tbl[pl.ds(cc * 32 + 16, 16)] = jnp.maximum(t1, r1)
            return carry

        lax.fori_loop(0, _PCNT // 16, group, 0)
        pltpu.sync_copy(tbl, out_h.at[pl.ds(wid * _TBLP, _TBLP)])

    return kern(x5f, cl1)


def kernel(x, pos, edge_index,
           W1, b1, gamma1, beta1,
           W2, b2, gamma2, beta2,
           W3, b3, gamma3, beta3,
           W4, b4, gamma4, beta4,
           W5, b5, gamma5, beta5,
           W6, b6, gamma6, beta6,
           W7, b7, gamma7, beta7,
           fcW):
    src1 = edge_index[0]
    dst1 = edge_index[1]
    zeros2d = jnp.zeros((1024, 16), jnp.float32)
    zeros1d = jnp.zeros((2048,), jnp.float32)

    cl2 = _cl_from_pos(pos)
    cl = cl2.reshape(N)

    deg0, deg1, h0, h1 = _sc_preprocess(src1, dst1, cl, zeros1d)
    h0 = h0[:NC1 * HP].reshape(NC1, HP)
    h1 = h1[:NC1 * HP].reshape(NC1, HP)
    w1p = jnp.pad(W1, ((0, 0), (0, 8)))
    dinv, z = _tc1(deg0.reshape(N, 1), deg1.reshape(N, 1), x, w1p)

    stats16 = _make_stats(1)
    stats32 = _make_stats(2)
    epi_1 = _make_epilogue(16, 8, 16, residual=False, matmul=True)
    epi_mid = _make_epilogue(16, 16, 16, residual=False, matmul=True)
    epi_res = _make_epilogue(16, 16, 32, residual=True, matmul=True)
    epi_5 = _make_epilogue(32, 32, 0, residual=False, matmul=False)

    g = lambda a: a.reshape(1, -1)

    parts = _sc_aggregate(z, src1, dst1, zeros2d)
    p, st = stats16(parts, z, dinv)
    x1, z = epi_1(p, st, g(gamma1), g(beta1), W2, dinv)
    parts = _sc_aggregate(z, src1, dst1, zeros2d)
    p, st = stats16(parts, z, dinv)
    x2, z = epi_mid(p, st, g(gamma2), g(beta2), W3, dinv)
    parts = _sc_aggregate(z, src1, dst1, zeros2d)
    p, st = stats16(parts, z, dinv)
    x3, z = epi_mid(p, st, g(gamma3), g(beta3), W4, dinv)
    parts = _sc_aggregate(z, src1, dst1, zeros2d)
    p, st = stats16(parts, z, dinv)
    x4, z5 = epi_res(p, st, g(gamma4), g(beta4), W5, dinv, res=x2)
    parts_a = _sc_aggregate(z5[:, :16], src1, dst1, zeros2d)
    parts_b = _sc_aggregate(z5[:, 16:], src1, dst1, zeros2d)
    p, st = stats32(list(parts_a) + list(parts_b), z5, dinv)
    x5 = epi_5(p, st, g(gamma5), g(beta5))
    tflat = _sc_pool_max(x5.reshape(N * 32), cl)
    tables = tflat.reshape(_NW, _TBLP)[:, :NC1 * 32].reshape(_NW, NC1, 32)
    fcw3 = fcW.reshape(NC7, 32, 2)
    return _final(h0, h1, tables, W6, g(gamma6), g(beta6), W7, g(gamma7),
                  g(beta7), fcw3)


# speedup vs baseline: 53.5127x; 1.0778x over previous
import functools

import jax
import jax.numpy as jnp
from jax import lax
from jax.experimental import pallas as pl
from jax.experimental.pallas import tpu as pltpu
from jax.experimental.pallas import tpu_sc as plsc

N = 100000
E = 1600000
PX, PY = 16.0, 12.0
NX, NY = 15, 15
NC1 = NX * NY
HP = 232
P7X, P7Y = 60.0, 45.0
NC7 = 16
EPS = 1e-5

BR = 2000
GRID = N // BR


def _cl_body(pos_ref, cl_ref):
    p = pos_ref[...]
    cx = jnp.clip(jnp.floor(p[:, 0:1] / PX), 0, NX - 1)
    cy = jnp.clip(jnp.floor(p[:, 1:2] / PY), 0, NY - 1)
    cl_ref[...] = (cx * NY + cy).astype(jnp.int32)


def _cl_from_pos(pos):
    return pl.pallas_call(
        _cl_body,
        grid=(GRID,),
        in_specs=[pl.BlockSpec((BR, 2), lambda i: (i, 0))],
        out_specs=pl.BlockSpec((BR, 1), lambda i: (i, 0)),
        out_shape=jax.ShapeDtypeStruct((N, 1), jnp.int32),
    )(pos)


def _tc1_body(d0_ref, d1_ref, x_ref, w1_ref, dinv_ref, z1_ref):
    deg = d0_ref[...] + d1_ref[...] + 1.0
    dinv = lax.rsqrt(deg)
    dinv_ref[...] = dinv
    z1_ref[...] = (x_ref[...] * dinv) * w1_ref[...]


def _tc1(deg0, deg1, x, w1p):
    return pl.pallas_call(
        _tc1_body,
        grid=(GRID,),
        in_specs=[
            pl.BlockSpec((BR, 1), lambda i: (i, 0)),
            pl.BlockSpec((BR, 1), lambda i: (i, 0)),
            pl.BlockSpec((BR, 1), lambda i: (i, 0)),
            pl.BlockSpec((1, 16), lambda i: (0, 0)),
        ],
        out_specs=[
            pl.BlockSpec((BR, 1), lambda i: (i, 0)),
            pl.BlockSpec((BR, 16), lambda i: (i, 0)),
        ],
        out_shape=[
            jax.ShapeDtypeStruct((N, 1), jnp.float32),
            jax.ShapeDtypeStruct((N, 16), jnp.float32),
        ],
    )(deg0, deg1, x, w1p)


def _make_stats(nparts):
    C = 16 * nparts

    def body(*refs):
        part_refs = refs[:2 * nparts]
        z_ref, dinv_ref, p_ref, stats_ref = refs[2 * nparts:]
        agg = jnp.concatenate(
            [part_refs[2 * i][...] + part_refs[2 * i + 1][...]
             for i in range(nparts)], axis=1)
        p = dinv_ref[...] * (agg + z_ref[...])
        p_ref[...] = p
        s1 = jnp.sum(p, axis=0, keepdims=True)
        s2 = jnp.sum(p * p, axis=0, keepdims=True)
        upd = jnp.concatenate([s1, s2, jnp.zeros((6, C), jnp.float32)], axis=0)

        @pl.when(pl.program_id(0) == 0)
        def _():
            stats_ref[...] = jnp.zeros_like(stats_ref)

        stats_ref[...] += upd

    def call(parts, z, dinv):
        return pl.pallas_call(
            body,
            grid=(GRID,),
            in_specs=[pl.BlockSpec((BR, 16), lambda i: (i, 0))] * (2 * nparts)
            + [
                pl.BlockSpec((BR, C), lambda i: (i, 0)),
                pl.BlockSpec((BR, 1), lambda i: (i, 0)),
            ],
            out_specs=[
                pl.BlockSpec((BR, C), lambda i: (i, 0)),
                pl.BlockSpec((8, C), lambda i: (0, 0)),
            ],
            out_shape=[
                jax.ShapeDtypeStruct((N, C), jnp.float32),
                jax.ShapeDtypeStruct((8, C), jnp.float32),
            ],
        )(*parts, z, dinv)

    return call


def _make_epilogue(Cp, Cin, Cout, residual, matmul):
    def body(*refs):
        it = iter(refs)
        p_ref = next(it)
        stats_ref = next(it)
        g_ref = next(it)
        b_ref = next(it)
        w_ref = next(it) if matmul else None
        dinv_ref = next(it) if matmul else None
        res_ref = next(it) if residual else None
        x_ref = next(it)
        z_ref = next(it) if matmul else None

        p = p_ref[...][:, :Cin]
        mean = stats_ref[0:1, :Cin] / N
        var = stats_ref[1:2, :Cin] / N - mean * mean
        x = (p - mean) * lax.rsqrt(var + EPS) * g_ref[...] + b_ref[...]
        x = jnp.maximum(x, 0.0)
        if residual:
            x = x + res_ref[...]
        x_ref[...] = x
        if matmul:
            z_ref[...] = (
                jnp.dot(x, w_ref[...], preferred_element_type=jnp.float32)
                * dinv_ref[...]
            )

    def call(p, stats, gamma, beta, W=None, dinv=None, res=None):
        in_specs = [
            pl.BlockSpec((BR, Cp), lambda i: (i, 0)),
            pl.BlockSpec((8, Cp), lambda i: (0, 0)),
            pl.BlockSpec((1, Cin), lambda i: (0, 0)),
            pl.BlockSpec((1, Cin), lambda i: (0, 0)),
        ]
        args = [p, stats, gamma, beta]
        if matmul:
            in_specs.append(pl.BlockSpec((Cin, Cout), lambda i: (0, 0)))
            in_specs.append(pl.BlockSpec((BR, 1), lambda i: (i, 0)))
            args += [W, dinv]
        if residual:
            in_specs.append(pl.BlockSpec((BR, Cin), lambda i: (i, 0)))
            args.append(res)
        out_specs = [pl.BlockSpec((BR, Cin), lambda i: (i, 0))]
        out_shape = [jax.ShapeDtypeStruct((N, Cin), jnp.float32)]
        if matmul:
            out_specs.append(pl.BlockSpec((BR, Cout), lambda i: (i, 0)))
            out_shape.append(jax.ShapeDtypeStruct((N, Cout), jnp.float32))
        res_ = pl.pallas_call(
            body,
            grid=(GRID,),
            in_specs=in_specs,
            out_specs=out_specs,
            out_shape=out_shape,
        )(*args)
        return res_ if matmul else res_[0]

    return call


def _final_body(h0_ref, h1_ref, tbl_ref, w6_ref, g6_ref, b6_ref, w7_ref,
                g7_ref, b7_ref, fcw_ref, out_ref):
    cm = h0_ref[...] + h1_ref[...]
    cm = cm[:, :NC1]
    rows = lax.broadcasted_iota(jnp.int32, (NC1, NC1), 0)
    cols = lax.broadcasted_iota(jnp.int32, (NC1, NC1), 1)
    cm = cm + jnp.where(rows == cols, 1.0, 0.0)
    degp = jnp.sum(cm, axis=1, keepdims=True)
    dp = lax.rsqrt(degp)

    xp = jnp.max(tbl_ref[...], axis=0)
    xp = jnp.where(jnp.isfinite(xp), xp, 0.0)

    def bn_relu(p, g, b):
        mean = jnp.mean(p, axis=0, keepdims=True)
        var = jnp.mean(p * p, axis=0, keepdims=True) - mean * mean
        return jnp.maximum((p - mean) * lax.rsqrt(var + EPS) * g + b, 0.0)

    y = jnp.dot(xp, w6_ref[...], preferred_element_type=jnp.float32)
    p = dp * jnp.dot(cm, dp * y, preferred_element_type=jnp.float32)
    x = bn_relu(p, g6_ref[...], b6_ref[...])
    y = jnp.dot(x, w7_ref[...], preferred_element_type=jnp.float32)
    p = dp * jnp.dot(cm, dp * y, preferred_element_type=jnp.float32)
    x = bn_relu(p, g7_ref[...], b7_ref[...]) + xp

    ids = lax.broadcasted_iota(jnp.int32, (NC1, 1), 0)
    r = ids // NY
    c = ids % NY
    c7x = jnp.clip(jnp.floor((r.astype(jnp.float32) + 0.5) * PX / P7X), 0, 3)
    c7y = jnp.clip(jnp.floor((c.astype(jnp.float32) + 0.5) * PY / P7Y), 0, 3)
    c7 = (c7x * 4 + c7y).astype(jnp.int32)
    acc = jnp.zeros((1, 2), jnp.float32)
    for k in range(NC7):
        mk = jnp.max(jnp.where(c7 == k, x, -jnp.inf), axis=0, keepdims=True)
        mk = jnp.where(jnp.isfinite(mk), mk, 0.0)
        acc = acc + jnp.dot(mk, fcw_ref[k], preferred_element_type=jnp.float32)
    out_ref[...] = acc


def _final(h0, h1, tables, W6, g6, b6, W7, g7, b7, fcw3):
    return pl.pallas_call(
        _final_body,
        out_shape=jax.ShapeDtypeStruct((1, 2), jnp.float32),
    )(h0, h1, tables, W6, g6, b6, W7, g7, b7, fcw3)


_SC_MESH = plsc.VectorSubcoreMesh(core_axis_name="c", subcore_axis_name="s")
_SC_PARAMS = pltpu.CompilerParams(use_tc_tiling_on_sc=False)
_NCORE, _NSUB, _NW = 2, 16, 32
_B = 128
_ROWS = E // _B
_RPW = _ROWS // _NW
_LEFT = _ROWS - _NW * _RPW
_NT = 6144
_NEXTRA = N - _NSUB * _NT
_HISTP = 52224
_HT = 3200
_HEXTRA = _HISTP - _NSUB * _HT


_AB = 512
_AROWS = E // _AB
_AMAIN = 96
_ATAIL = _AROWS - _NW * _AMAIN
_ANB = 3
_AK = 2


def _sc_aggregate(z, esd, zeros2d):
    @functools.partial(
        pl.kernel,
        out_type=[
            jax.ShapeDtypeStruct((N, 16), jnp.float32),
            jax.ShapeDtypeStruct((N, 16), jnp.float32),
        ],
        mesh=_SC_MESH,
        compiler_params=_SC_PARAMS,
        scratch_types=[
            [pltpu.VMEM((2 * _AB,), jnp.int32)] * _ANB,
            [pltpu.VMEM((_AB, 16), jnp.float32)] * _ANB,
            pltpu.VMEM_SHARED((N, 16), jnp.float32),
            [pltpu.SemaphoreType.DMA] * _ANB,
            [pltpu.SemaphoreType.DMA] * _ANB,
        ],
    )
    def kern(z_h, esd_h, zeros_h, out0_h, out1_h, sdbs, rbs,
             acc, gsems, ssems):
        cid = lax.axis_index("c")
        sid = lax.axis_index("s")
        wid = cid * _NSUB + sid

        for j in range(6):
            pltpu.sync_copy(zeros_h, acc.at[pl.ds(sid * _NT + j * 1024, 1024)])

        @pl.when(sid == 0)
        def _():
            pltpu.sync_copy(zeros_h, acc.at[pl.ds(_NSUB * _NT, 1024)])
            pltpu.sync_copy(zeros_h.at[pl.ds(0, _NEXTRA - 1024)],
                            acc.at[pl.ds(_NSUB * _NT + 1024, _NEXTRA - 1024)])

        plsc.subcore_barrier()

        def off(j):
            return (wid + j * _NW) * 2 * _AB

        def src_idx(b):
            return sdbs[b].at[pl.ds(0, _AB)]

        def dst_idx(b):
            return sdbs[b].at[pl.ds(_AB, _AB)]

        def fetch(j, b):
            pltpu.sync_copy(esd_h.at[pl.ds(off(j), 2 * _AB)], sdbs[b])
            pltpu.async_copy(z_h.at[src_idx(b)], rbs[b], gsems[b])

        for t in range(_AK):
            fetch(t, t)

        def step(it, carry):
            k = it * _ANB
            for b in range(_ANB):
                j = k + b
                pltpu.make_async_copy(z_h.at[src_idx(b)], rbs[b],
                                      gsems[b]).wait()
                pltpu.async_copy(rbs[b], acc.at[dst_idx(b)], ssems[b],
                                 add=True)
                jp = j + _AK
                bp = (b + _AK) % _ANB

                @pl.when(jp < _AMAIN)
                def _():
                    @pl.when(jp >= _ANB)
                    def _():
                        pltpu.make_async_copy(rbs[bp], acc.at[dst_idx(bp)],
                                              ssems[bp]).wait()

                    fetch(jp, bp)

            return carry

        lax.fori_loop(0, _AMAIN // _ANB, step, 0)
        for b in range(_ANB):
            pltpu.make_async_copy(rbs[b], acc.at[dst_idx(b)], ssems[b]).wait()

        def tail_row(r):
            pltpu.sync_copy(esd_h.at[pl.ds(r * 2 * _AB, 2 * _AB)], sdbs[0])
            pltpu.async_copy(z_h.at[src_idx(0)], rbs[0], gsems[0]).wait()
            pltpu.sync_copy(rbs[0], acc.at[dst_idx(0)], add=True)

        tail_row(_NW * _AMAIN + wid)

        @pl.when(wid < _ATAIL - _NW)
        def _():
            tail_row(_NW * _AMAIN + _NW + wid)

        plsc.subcore_barrier()

        def copy_out(r0, nrows):
            @pl.when(cid == 0)
            def _():
                pltpu.sync_copy(acc.at[pl.ds(r0, nrows)],
                                out0_h.at[pl.ds(r0, nrows)])

            @pl.when(cid == 1)
            def _():
                pltpu.sync_copy(acc.at[pl.ds(r0, nrows)],
                                out1_h.at[pl.ds(r0, nrows)])

        for j in range(3):
            copy_out(sid * _NT + j * 2048, 2048)

        @pl.when(sid == 0)
        def _():
            copy_out(_NSUB * _NT, _NEXTRA)

    return kern(z, esd, zeros2d)


def _sc_preprocess(esd, cl1, zeros1d):
    @functools.partial(
        pl.kernel,
        out_type=[
            jax.ShapeDtypeStruct((N,), jnp.float32),
            jax.ShapeDtypeStruct((N,), jnp.float32),
            jax.ShapeDtypeStruct((_HISTP,), jnp.float32),
            jax.ShapeDtypeStruct((_HISTP,), jnp.float32),
        ],
        mesh=_SC_MESH,
        compiler_params=_SC_PARAMS,
        scratch_types=[
            [pltpu.VMEM((2 * _AB,), jnp.int32)] * _ANB,
            [pltpu.VMEM((2 * _AB,), jnp.int32)] * _ANB,
            pltpu.VMEM((_AB,), jnp.int32),
            pltpu.VMEM((_AB,), jnp.float32),
            pltpu.VMEM_SHARED((N,), jnp.float32),
            pltpu.VMEM_SHARED((_HISTP,), jnp.float32),
            [pltpu.SemaphoreType.DMA] * _ANB,
        ],
    )
    def kern(esd_h, cl_h, zeros_h, deg0_h, deg1_h, h0_h, h1_h, sdbs,
             clbs, binb, ones, dega, hista, gss):
        cid = lax.axis_index("c")
        sid = lax.axis_index("s")
        wid = cid * _NSUB + sid
        onev = jnp.ones((16,), jnp.float32)
        for i in range(_AB // 16):
            ones[pl.ds(i * 16, 16)] = onev

        for k in range(3):
            pltpu.sync_copy(zeros_h,
                            dega.at[pl.ds(sid * _NT + k * 2048, 2048)])

        @pl.when(sid == 0)
        def _():
            pltpu.sync_copy(zeros_h.at[pl.ds(0, _NEXTRA)],
                            dega.at[pl.ds(_NSUB * _NT, _NEXTRA)])

        pltpu.sync_copy(zeros_h, hista.at[pl.ds(sid * _HT, 2048)])
        pltpu.sync_copy(zeros_h.at[pl.ds(0, _HT - 2048)],
                        hista.at[pl.ds(sid * _HT + 2048, _HT - 2048)])

        @pl.when(sid == 0)
        def _():
            pltpu.sync_copy(zeros_h.at[pl.ds(0, _HEXTRA)],
                            hista.at[pl.ds(_NSUB * _HT, _HEXTRA)])

        plsc.subcore_barrier()

        def off(j):
            return (wid + j * _NW) * 2 * _AB

        def fetch(j, b):
            pltpu.sync_copy(esd_h.at[pl.ds(off(j), 2 * _AB)], sdbs[b])
            pltpu.async_copy(cl_h.at[sdbs[b]], clbs[b], gss[b])

        for t in range(_AK):
            fetch(t, t)

        def body(b):
            for i in range(_AB // 16):
                s = clbs[b][pl.ds(i * 16, 16)]
                dd = clbs[b][pl.ds(_AB + i * 16, 16)]
                binb[pl.ds(i * 16, 16)] = dd * HP + s
            pltpu.sync_copy(ones, hista.at[binb], add=True)
            pltpu.sync_copy(ones, dega.at[sdbs[b].at[pl.ds(_AB, _AB)]],
                            add=True)

        def step(it, carry):
            k = it * _ANB
            for b in range(_ANB):
                j = k + b
                pltpu.make_async_copy(cl_h.at[sdbs[b]], clbs[b],
                                      gss[b]).wait()
                body(b)
                jp = j + _AK
                bp = (b + _AK) % _ANB

                @pl.when(jp < _AMAIN)
                def _():
                    fetch(jp, bp)

            return carry

        lax.fori_loop(0, _AMAIN // _ANB, step, 0)

        def tail_row(r):
            pltpu.sync_copy(esd_h.at[pl.ds(r * 2 * _AB, 2 * _AB)], sdbs[0])
            pltpu.async_copy(cl_h.at[sdbs[0]], clbs[0], gss[0]).wait()
            body(0)

        tail_row(_NW * _AMAIN + wid)

        @pl.when(wid < _ATAIL - _NW)
        def _():
            tail_row(_NW * _AMAIN + _NW + wid)

        plsc.subcore_barrier()

        def out_chunk(acc_ref, o0_h, o1_h, o, n):
            @pl.when(cid == 0)
            def _():
                pltpu.sync_copy(acc_ref.at[pl.ds(o, n)],
                                o0_h.at[pl.ds(o, n)])

            @pl.when(cid == 1)
            def _():
                pltpu.sync_copy(acc_ref.at[pl.ds(o, n)],
                                o1_h.at[pl.ds(o, n)])

        for k in range(3):
            out_chunk(dega, deg0_h, deg1_h, sid * _NT + k * 2048, 2048)

        @pl.when(sid == 0)
        def _():
            out_chunk(dega, deg0_h, deg1_h, _NSUB * _NT, _NEXTRA)

        out_chunk(hista, h0_h, h1_h, sid * _HT, 2048)
        out_chunk(hista, h0_h, h1_h, sid * _HT + 2048, _HT - 2048)

        @pl.when(sid == 0)
        def _():
            out_chunk(hista, h0_h, h1_h, _NSUB * _HT, _HEXTRA)

    return kern(esd, cl1, zeros1d)


_PSTRIDE = 3072
_PCNT = 3360
_TBLP = 7296


def _sc_pool_max(x5f, cl1):
    @functools.partial(
        pl.kernel,
        out_type=jax.ShapeDtypeStruct((_NW * _TBLP,), jnp.float32),
        mesh=_SC_MESH,
        compiler_params=_SC_PARAMS,
        scratch_types=[
            pltpu.VMEM((_PCNT * 32,), jnp.float32),
            pltpu.VMEM((_PCNT,), jnp.int32),
            pltpu.VMEM((_TBLP,), jnp.float32),
        ],
    )
    def kern(x_h, cl_h, out_h, xb, clb, tbl):
        cid = lax.axis_index("c")
        sid = lax.axis_index("s")
        wid = cid * _NSUB + sid
        neg = jnp.full((16,), -jnp.inf, jnp.float32)

        def trow(i, carry):
            tbl[pl.ds(i * 16, 16)] = neg
            return carry

        lax.fori_loop(0, _TBLP // 16, trow, 0)

        base = jnp.where(wid == _NW - 1, N - _PCNT,
                         (wid * 3125) // 128 * 128)
        pltpu.sync_copy(x_h.at[pl.ds(base * 32, _PCNT * 32)], xb)
        pltpu.sync_copy(cl_h.at[pl.ds(base, _PCNT)], clb)

        def group(g, carry):
            cvec = clb[pl.ds(g * 16, 16)]
            for b in range(16):
                i = g * 16 + b
                cc = cvec[b]
                r0 = xb[pl.ds(i * 32, 16)]
                r1 = xb[pl.ds(i * 32 + 16, 16)]
                t0 = tbl[pl.ds(cc * 32, 16)]
                t1 = tbl[pl.ds(cc * 32 + 16, 16)]
                tbl[pl.ds(cc * 32, 16)] = jnp.maximum(t0, r0)
                tbl[pl.ds(cc * 32 + 16, 16)] = jnp.maximum(t1, r1)
            return carry

        lax.fori_loop(0, _PCNT // 16, group, 0)
        pltpu.sync_copy(tbl, out_h.at[pl.ds(wid * _TBLP, _TBLP)])

    return kern(x5f, cl1)


def kernel(x, pos, edge_index,
           W1, b1, gamma1, beta1,
           W2, b2, gamma2, beta2,
           W3, b3, gamma3, beta3,
           W4, b4, gamma4, beta4,
           W5, b5, gamma5, beta5,
           W6, b6, gamma6, beta6,
           W7, b7, gamma7, beta7,
           fcW):
    esd = edge_index.reshape(2, _AROWS, _AB).transpose(1, 0, 2).reshape(-1)
    zeros2d = jnp.zeros((1024, 16), jnp.float32)
    zeros1d = jnp.zeros((2048,), jnp.float32)

    cl2 = _cl_from_pos(pos)
    cl = cl2.reshape(N)

    deg0, deg1, h0, h1 = _sc_preprocess(esd, cl, zeros1d)
    h0 = h0[:NC1 * HP].reshape(NC1, HP)
    h1 = h1[:NC1 * HP].reshape(NC1, HP)
    w1p = jnp.pad(W1, ((0, 0), (0, 8)))
    dinv, z = _tc1(deg0.reshape(N, 1), deg1.reshape(N, 1), x, w1p)

    stats16 = _make_stats(1)
    stats32 = _make_stats(2)
    epi_1 = _make_epilogue(16, 8, 16, residual=False, matmul=True)
    epi_mid = _make_epilogue(16, 16, 16, residual=False, matmul=True)
    epi_res = _make_epilogue(16, 16, 32, residual=True, matmul=True)
    epi_5 = _make_epilogue(32, 32, 0, residual=False, matmul=False)

    g = lambda a: a.reshape(1, -1)

    parts = _sc_aggregate(z, esd, zeros2d)
    p, st = stats16(parts, z, dinv)
    x1, z = epi_1(p, st, g(gamma1), g(beta1), W2, dinv)
    parts = _sc_aggregate(z, esd, zeros2d)
    p, st = stats16(parts, z, dinv)
    x2, z = epi_mid(p, st, g(gamma2), g(beta2), W3, dinv)
    parts = _sc_aggregate(z, esd, zeros2d)
    p, st = stats16(parts, z, dinv)
    x3, z = epi_mid(p, st, g(gamma3), g(beta3), W4, dinv)
    parts = _sc_aggregate(z, esd, zeros2d)
    p, st = stats16(parts, z, dinv)
    x4, z5 = epi_res(p, st, g(gamma4), g(beta4), W5, dinv, res=x2)
    parts_a = _sc_aggregate(z5[:, :16], esd, zeros2d)
    parts_b = _sc_aggregate(z5[:, 16:], esd, zeros2d)
    p, st = stats32(list(parts_a) + list(parts_b), z5, dinv)
    x5 = epi_5(p, st, g(gamma5), g(beta5))
    tflat = _sc_pool_max(x5.reshape(N * 32), cl)
    tables = tflat.reshape(_NW, _TBLP)[:, :NC1 * 32].reshape(_NW, NC1, 32)
    fcw3 = fcW.reshape(NC7, 32, 2)
    return _final(h0, h1, tables, W6, g(gamma6), g(beta6), W7, g(gamma7),
                  g(beta7), fcw3)


# speedup vs baseline: 54.7355x; 1.0229x over previous
import functools

import jax
import jax.numpy as jnp
from jax import lax
from jax.experimental import pallas as pl
from jax.experimental.pallas import tpu as pltpu
from jax.experimental.pallas import tpu_sc as plsc

N = 100000
E = 1600000
PX, PY = 16.0, 12.0
NX, NY = 15, 15
NC1 = NX * NY
HP = 232
P7X, P7Y = 60.0, 45.0
NC7 = 16
EPS = 1e-5

BR = 4000
GRID = N // BR


def _cl_body(pos_ref, cl_ref):
    p = pos_ref[...]
    cx = jnp.clip(jnp.floor(p[:, 0:1] / PX), 0, NX - 1)
    cy = jnp.clip(jnp.floor(p[:, 1:2] / PY), 0, NY - 1)
    cl_ref[...] = (cx * NY + cy).astype(jnp.int32)


def _cl_from_pos(pos):
    return pl.pallas_call(
        _cl_body,
        grid=(GRID,),
        in_specs=[pl.BlockSpec((BR, 2), lambda i: (i, 0))],
        out_specs=pl.BlockSpec((BR, 1), lambda i: (i, 0)),
        out_shape=jax.ShapeDtypeStruct((N, 1), jnp.int32),
    )(pos)


def _tc1_body(d0_ref, d1_ref, x_ref, w1_ref, dinv_ref, z1_ref):
    deg = d0_ref[...] + d1_ref[...] + 1.0
    dinv = lax.rsqrt(deg)
    dinv_ref[...] = dinv
    z1_ref[...] = (x_ref[...] * dinv) * w1_ref[...]


def _tc1(deg0, deg1, x, w1p):
    return pl.pallas_call(
        _tc1_body,
        grid=(GRID,),
        in_specs=[
            pl.BlockSpec((BR, 1), lambda i: (i, 0)),
            pl.BlockSpec((BR, 1), lambda i: (i, 0)),
            pl.BlockSpec((BR, 1), lambda i: (i, 0)),
            pl.BlockSpec((1, 16), lambda i: (0, 0)),
        ],
        out_specs=[
            pl.BlockSpec((BR, 1), lambda i: (i, 0)),
            pl.BlockSpec((BR, 16), lambda i: (i, 0)),
        ],
        out_shape=[
            jax.ShapeDtypeStruct((N, 1), jnp.float32),
            jax.ShapeDtypeStruct((N, 16), jnp.float32),
        ],
    )(deg0, deg1, x, w1p)


def _make_stats(nparts):
    C = 16 * nparts

    def body(*refs):
        part_refs = refs[:2 * nparts]
        z_refs = refs[2 * nparts:3 * nparts]
        dinv_ref, p_ref, stats_ref = refs[3 * nparts:]
        agg = jnp.concatenate(
            [part_refs[2 * i][...] + part_refs[2 * i + 1][...]
             for i in range(nparts)], axis=1)
        z = jnp.concatenate([r[...] for r in z_refs], axis=1)
        p = dinv_ref[...] * (agg + z)
        p_ref[...] = p
        s1 = jnp.sum(p, axis=0, keepdims=True)
        s2 = jnp.sum(p * p, axis=0, keepdims=True)
        upd = jnp.concatenate([s1, s2, jnp.zeros((6, C), jnp.float32)], axis=0)

        @pl.when(pl.program_id(0) == 0)
        def _():
            stats_ref[...] = jnp.zeros_like(stats_ref)

        stats_ref[...] += upd

    def call(parts, zs, dinv):
        return pl.pallas_call(
            body,
            grid=(GRID,),
            in_specs=[pl.BlockSpec((BR, 16), lambda i: (i, 0))] * (3 * nparts)
            + [
                pl.BlockSpec((BR, 1), lambda i: (i, 0)),
            ],
            out_specs=[
                pl.BlockSpec((BR, C), lambda i: (i, 0)),
                pl.BlockSpec((8, C), lambda i: (0, 0)),
            ],
            out_shape=[
                jax.ShapeDtypeStruct((N, C), jnp.float32),
                jax.ShapeDtypeStruct((8, C), jnp.float32),
            ],
        )(*parts, *zs, dinv)

    return call


def _make_epilogue(Cp, Cin, Cout, residual, matmul):
    def body(*refs):
        it = iter(refs)
        p_ref = next(it)
        stats_ref = next(it)
        g_ref = next(it)
        b_ref = next(it)
        w_ref = next(it) if matmul else None
        dinv_ref = next(it) if matmul else None
        res_ref = next(it) if residual else None
        x_ref = next(it)
        z_ref = next(it) if matmul else None

        p = p_ref[...][:, :Cin]
        mean = stats_ref[0:1, :Cin] / N
        var = stats_ref[1:2, :Cin] / N - mean * mean
        x = (p - mean) * lax.rsqrt(var + EPS) * g_ref[...] + b_ref[...]
        x = jnp.maximum(x, 0.0)
        if residual:
            x = x + res_ref[...]
        x_ref[...] = x
        if matmul:
            zz = (
                jnp.dot(x, w_ref[...], preferred_element_type=jnp.float32)
                * dinv_ref[...]
            )
            if Cout > 16:
                z_ref[...] = zz[:, :16]
                z2_ref = next(it)
                z2_ref[...] = zz[:, 16:]
            else:
                z_ref[...] = zz

    def call(p, stats, gamma, beta, W=None, dinv=None, res=None):
        in_specs = [
            pl.BlockSpec((BR, Cp), lambda i: (i, 0)),
            pl.BlockSpec((8, Cp), lambda i: (0, 0)),
            pl.BlockSpec((1, Cin), lambda i: (0, 0)),
            pl.BlockSpec((1, Cin), lambda i: (0, 0)),
        ]
        args = [p, stats, gamma, beta]
        if matmul:
            in_specs.append(pl.BlockSpec((Cin, Cout), lambda i: (0, 0)))
            in_specs.append(pl.BlockSpec((BR, 1), lambda i: (i, 0)))
            args += [W, dinv]
        if residual:
            in_specs.append(pl.BlockSpec((BR, Cin), lambda i: (i, 0)))
            args.append(res)
        out_specs = [pl.BlockSpec((BR, Cin), lambda i: (i, 0))]
        out_shape = [jax.ShapeDtypeStruct((N, Cin), jnp.float32)]
        if matmul:
            nz = 2 if Cout > 16 else 1
            for _ in range(nz):
                out_specs.append(
                    pl.BlockSpec((BR, Cout // nz), lambda i: (i, 0)))
                out_shape.append(
                    jax.ShapeDtypeStruct((N, Cout // nz), jnp.float32))
        res_ = pl.pallas_call(
            body,
            grid=(GRID,),
            in_specs=in_specs,
            out_specs=out_specs,
            out_shape=out_shape,
        )(*args)
        return res_ if matmul else res_[0]

    return call


def _final_body(h0_ref, h1_ref, tbl_ref, w6_ref, g6_ref, b6_ref, w7_ref,
                g7_ref, b7_ref, fcw_ref, out_ref):
    cm = h0_ref[...] + h1_ref[...]
    cm = cm[:, :NC1]
    rows = lax.broadcasted_iota(jnp.int32, (NC1, NC1), 0)
    cols = lax.broadcasted_iota(jnp.int32, (NC1, NC1), 1)
    cm = cm + jnp.where(rows == cols, 1.0, 0.0)
    degp = jnp.sum(cm, axis=1, keepdims=True)
    dp = lax.rsqrt(degp)

    xp = jnp.max(tbl_ref[...], axis=0)
    xp = jnp.where(jnp.isfinite(xp), xp, 0.0)

    def bn_relu(p, g, b):
        mean = jnp.mean(p, axis=0, keepdims=True)
        var = jnp.mean(p * p, axis=0, keepdims=True) - mean * mean
        return jnp.maximum((p - mean) * lax.rsqrt(var + EPS) * g + b, 0.0)

    y = jnp.dot(xp, w6_ref[...], preferred_element_type=jnp.float32)
    p = dp * jnp.dot(cm, dp * y, preferred_element_type=jnp.float32)
    x = bn_relu(p, g6_ref[...], b6_ref[...])
    y = jnp.dot(x, w7_ref[...], preferred_element_type=jnp.float32)
    p = dp * jnp.dot(cm, dp * y, preferred_element_type=jnp.float32)
    x = bn_relu(p, g7_ref[...], b7_ref[...]) + xp

    ids = lax.broadcasted_iota(jnp.int32, (NC1, 1), 0)
    r = ids // NY
    c = ids % NY
    c7x = jnp.clip(jnp.floor((r.astype(jnp.float32) + 0.5) * PX / P7X), 0, 3)
    c7y = jnp.clip(jnp.floor((c.astype(jnp.float32) + 0.5) * PY / P7Y), 0, 3)
    c7 = (c7x * 4 + c7y).astype(jnp.int32)
    acc = jnp.zeros((1, 2), jnp.float32)
    for k in range(NC7):
        mk = jnp.max(jnp.where(c7 == k, x, -jnp.inf), axis=0, keepdims=True)
        mk = jnp.where(jnp.isfinite(mk), mk, 0.0)
        acc = acc + jnp.dot(mk, fcw_ref[k], preferred_element_type=jnp.float32)
    out_ref[...] = acc


def _final(h0, h1, tables, W6, g6, b6, W7, g7, b7, fcw3):
    return pl.pallas_call(
        _final_body,
        out_shape=jax.ShapeDtypeStruct((1, 2), jnp.float32),
    )(h0, h1, tables, W6, g6, b6, W7, g7, b7, fcw3)


_SC_MESH = plsc.VectorSubcoreMesh(core_axis_name="c", subcore_axis_name="s")
_SC_PARAMS = pltpu.CompilerParams(use_tc_tiling_on_sc=False)
_NCORE, _NSUB, _NW = 2, 16, 32
_B = 128
_ROWS = E // _B
_RPW = _ROWS // _NW
_LEFT = _ROWS - _NW * _RPW
_NT = 6144
_NEXTRA = N - _NSUB * _NT
_HISTP = 52224
_HT = 3200
_HEXTRA = _HISTP - _NSUB * _HT


_AB = 512
_AROWS = E // _AB
_AMAIN = 96
_ATAIL = _AROWS - _NW * _AMAIN
_ANB = 3
_AK = 2


def _sc_aggregate(z, esd, zeros2d):
    @functools.partial(
        pl.kernel,
        out_type=[
            jax.ShapeDtypeStruct((N, 16), jnp.float32),
            jax.ShapeDtypeStruct((N, 16), jnp.float32),
        ],
        mesh=_SC_MESH,
        compiler_params=_SC_PARAMS,
        scratch_types=[
            [pltpu.VMEM((2 * _AB,), jnp.int32)] * _ANB,
            [pltpu.VMEM((_AB, 16), jnp.float32)] * _ANB,
            pltpu.VMEM_SHARED((N, 16), jnp.float32),
            [pltpu.SemaphoreType.DMA] * _ANB,
            [pltpu.SemaphoreType.DMA] * _ANB,
        ],
    )
    def kern(z_h, esd_h, zeros_h, out0_h, out1_h, sdbs, rbs,
             acc, gsems, ssems):
        cid = lax.axis_index("c")
        sid = lax.axis_index("s")
        wid = cid * _NSUB + sid

        for j in range(6):
            pltpu.sync_copy(zeros_h, acc.at[pl.ds(sid * _NT + j * 1024, 1024)])

        @pl.when(sid == 0)
        def _():
            pltpu.sync_copy(zeros_h, acc.at[pl.ds(_NSUB * _NT, 1024)])
            pltpu.sync_copy(zeros_h.at[pl.ds(0, _NEXTRA - 1024)],
                            acc.at[pl.ds(_NSUB * _NT + 1024, _NEXTRA - 1024)])

        plsc.subcore_barrier()

        def off(j):
            return (wid + j * _NW) * 2 * _AB

        def src_idx(b):
            return sdbs[b].at[pl.ds(0, _AB)]

        def dst_idx(b):
            return sdbs[b].at[pl.ds(_AB, _AB)]

        def fetch(j, b):
            pltpu.sync_copy(esd_h.at[pl.ds(off(j), 2 * _AB)], sdbs[b])
            pltpu.async_copy(z_h.at[src_idx(b)], rbs[b], gsems[b])

        for t in range(_AK):
            fetch(t, t)

        def step(it, carry):
            k = it * _ANB
            for b in range(_ANB):
                j = k + b
                pltpu.make_async_copy(z_h.at[src_idx(b)], rbs[b],
                                      gsems[b]).wait()
                pltpu.async_copy(rbs[b], acc.at[dst_idx(b)], ssems[b],
                                 add=True)
                jp = j + _AK
                bp = (b + _AK) % _ANB

                @pl.when(jp < _AMAIN)
                def _():
                    @pl.when(jp >= _ANB)
                    def _():
                        pltpu.make_async_copy(rbs[bp], acc.at[dst_idx(bp)],
                                              ssems[bp]).wait()

                    fetch(jp, bp)

            return carry

        lax.fori_loop(0, _AMAIN // _ANB, step, 0)
        for b in range(_ANB):
            pltpu.make_async_copy(rbs[b], acc.at[dst_idx(b)], ssems[b]).wait()

        def tail_row(r):
            pltpu.sync_copy(esd_h.at[pl.ds(r * 2 * _AB, 2 * _AB)], sdbs[0])
            pltpu.async_copy(z_h.at[src_idx(0)], rbs[0], gsems[0]).wait()
            pltpu.sync_copy(rbs[0], acc.at[dst_idx(0)], add=True)

        tail_row(_NW * _AMAIN + wid)

        @pl.when(wid < _ATAIL - _NW)
        def _():
            tail_row(_NW * _AMAIN + _NW + wid)

        plsc.subcore_barrier()

        def copy_out(r0, nrows):
            @pl.when(cid == 0)
            def _():
                pltpu.sync_copy(acc.at[pl.ds(r0, nrows)],
                                out0_h.at[pl.ds(r0, nrows)])

            @pl.when(cid == 1)
            def _():
                pltpu.sync_copy(acc.at[pl.ds(r0, nrows)],
                                out1_h.at[pl.ds(r0, nrows)])

        for j in range(3):
            copy_out(sid * _NT + j * 2048, 2048)

        @pl.when(sid == 0)
        def _():
            copy_out(_NSUB * _NT, _NEXTRA)

    return kern(z, esd, zeros2d)


def _sc_preprocess(esd, cl1, zeros1d):
    @functools.partial(
        pl.kernel,
        out_type=[
            jax.ShapeDtypeStruct((N,), jnp.float32),
            jax.ShapeDtypeStruct((N,), jnp.float32),
            jax.ShapeDtypeStruct((_HISTP,), jnp.float32),
            jax.ShapeDtypeStruct((_HISTP,), jnp.float32),
        ],
        mesh=_SC_MESH,
        compiler_params=_SC_PARAMS,
        scratch_types=[
            [pltpu.VMEM((2 * _AB,), jnp.int32)] * _ANB,
            [pltpu.VMEM((2 * _AB,), jnp.int32)] * _ANB,
            pltpu.VMEM((_AB,), jnp.int32),
            pltpu.VMEM((_AB,), jnp.float32),
            pltpu.VMEM_SHARED((N,), jnp.float32),
            pltpu.VMEM_SHARED((_HISTP,), jnp.float32),
            [pltpu.SemaphoreType.DMA] * _ANB,
        ],
    )
    def kern(esd_h, cl_h, zeros_h, deg0_h, deg1_h, h0_h, h1_h, sdbs,
             clbs, binb, ones, dega, hista, gss):
        cid = lax.axis_index("c")
        sid = lax.axis_index("s")
        wid = cid * _NSUB + sid
        onev = jnp.ones((16,), jnp.float32)
        for i in range(_AB // 16):
            ones[pl.ds(i * 16, 16)] = onev

        for k in range(3):
            pltpu.sync_copy(zeros_h,
                            dega.at[pl.ds(sid * _NT + k * 2048, 2048)])

        @pl.when(sid == 0)
        def _():
            pltpu.sync_copy(zeros_h.at[pl.ds(0, _NEXTRA)],
                            dega.at[pl.ds(_NSUB * _NT, _NEXTRA)])

        pltpu.sync_copy(zeros_h, hista.at[pl.ds(sid * _HT, 2048)])
        pltpu.sync_copy(zeros_h.at[pl.ds(0, _HT - 2048)],
                        hista.at[pl.ds(sid * _HT + 2048, _HT - 2048)])

        @pl.when(sid == 0)
        def _():
            pltpu.sync_copy(zeros_h.at[pl.ds(0, _HEXTRA)],
                            hista.at[pl.ds(_NSUB * _HT, _HEXTRA)])

        plsc.subcore_barrier()

        def off(j):
            return (wid + j * _NW) * 2 * _AB

        def fetch(j, b):
            pltpu.sync_copy(esd_h.at[pl.ds(off(j), 2 * _AB)], sdbs[b])
            pltpu.async_copy(cl_h.at[sdbs[b]], clbs[b], gss[b])

        for t in range(_AK):
            fetch(t, t)

        def body(b):
            for i in range(_AB // 16):
                s = clbs[b][pl.ds(i * 16, 16)]
                dd = clbs[b][pl.ds(_AB + i * 16, 16)]
                binb[pl.ds(i * 16, 16)] = dd * HP + s
            pltpu.sync_copy(ones, hista.at[binb], add=True)
            pltpu.sync_copy(ones, dega.at[sdbs[b].at[pl.ds(_AB, _AB)]],
                            add=True)

        def step(it, carry):
            k = it * _ANB
            for b in range(_ANB):
                j = k + b
                pltpu.make_async_copy(cl_h.at[sdbs[b]], clbs[b],
                                      gss[b]).wait()
                body(b)
                jp = j + _AK
                bp = (b + _AK) % _ANB

                @pl.when(jp < _AMAIN)
                def _():
                    fetch(jp, bp)

            return carry

        lax.fori_loop(0, _AMAIN // _ANB, step, 0)

        def tail_row(r):
            pltpu.sync_copy(esd_h.at[pl.ds(r * 2 * _AB, 2 * _AB)], sdbs[0])
            pltpu.async_copy(cl_h.at[sdbs[0]], clbs[0], gss[0]).wait()
            body(0)

        tail_row(_NW * _AMAIN + wid)

        @pl.when(wid < _ATAIL - _NW)
        def _():
            tail_row(_NW * _AMAIN + _NW + wid)

        plsc.subcore_barrier()

        def out_chunk(acc_ref, o0_h, o1_h, o, n):
            @pl.when(cid == 0)
            def _():
                pltpu.sync_copy(acc_ref.at[pl.ds(o, n)],
                                o0_h.at[pl.ds(o, n)])

            @pl.when(cid == 1)
            def _():
                pltpu.sync_copy(acc_ref.at[pl.ds(o, n)],
                                o1_h.at[pl.ds(o, n)])

        for k in range(3):
            out_chunk(dega, deg0_h, deg1_h, sid * _NT + k * 2048, 2048)

        @pl.when(sid == 0)
        def _():
            out_chunk(dega, deg0_h, deg1_h, _NSUB * _NT, _NEXTRA)

        out_chunk(hista, h0_h, h1_h, sid * _HT, 2048)
        out_chunk(hista, h0_h, h1_h, sid * _HT + 2048, _HT - 2048)

        @pl.when(sid == 0)
        def _():
            out_chunk(hista, h0_h, h1_h, _NSUB * _HT, _HEXTRA)

    return kern(esd, cl1, zeros1d)


_PSTRIDE = 3072
_PCNT = 3360
_TBLP = 7296


def _sc_pool_max(x5f, cl1):
    @functools.partial(
        pl.kernel,
        out_type=jax.ShapeDtypeStruct((_NW * _TBLP,), jnp.float32),
        mesh=_SC_MESH,
        compiler_params=_SC_PARAMS,
        scratch_types=[
            pltpu.VMEM((_PCNT * 32,), jnp.float32),
            pltpu.VMEM((_PCNT,), jnp.int32),
            pltpu.VMEM((_TBLP,), jnp.float32),
        ],
    )
    def kern(x_h, cl_h, out_h, xb, clb, tbl):
        cid = lax.axis_index("c")
        sid = lax.axis_index("s")
        wid = cid * _NSUB + sid
        neg = jnp.full((16,), -jnp.inf, jnp.float32)

        def trow(i, carry):
            tbl[pl.ds(i * 16, 16)] = neg
            return carry

        lax.fori_loop(0, _TBLP // 16, trow, 0)

        base = jnp.where(wid == _NW - 1, N - _PCNT,
                         (wid * 3125) // 128 * 128)
        pltpu.sync_copy(x_h.at[pl.ds(base * 32, _PCNT * 32)], xb)
        pltpu.sync_copy(cl_h.at[pl.ds(base, _PCNT)], clb)

        def group(g, carry):
            cvec = clb[pl.ds(g * 16, 16)]
            for b in range(16):
                i = g * 16 + b
                cc = cvec[b]
                r0 = xb[pl.ds(i * 32, 16)]
                r1 = xb[pl.ds(i * 32 + 16, 16)]
                t0 = tbl[pl.ds(cc * 32, 16)]
                t1 = tbl[pl.ds(cc * 32 + 16, 16)]
                tbl[pl.ds(cc * 32, 16)] = jnp.maximum(t0, r0)
                tbl[pl.ds(cc * 32 + 16, 16)] = jnp.maximum(t1, r1)
            return carry

        lax.fori_loop(0, _PCNT // 16, group, 0)
        pltpu.sync_copy(tbl, out_h.at[pl.ds(wid * _TBLP, _TBLP)])

    return kern(x5f, cl1)


def kernel(x, pos, edge_index,
           W1, b1, gamma1, beta1,
           W2, b2, gamma2, beta2,
           W3, b3, gamma3, beta3,
           W4, b4, gamma4, beta4,
           W5, b5, gamma5, beta5,
           W6, b6, gamma6, beta6,
           W7, b7, gamma7, beta7,
           fcW):
    esd = edge_index.reshape(2, _AROWS, _AB).transpose(1, 0, 2).reshape(-1)
    zeros2d = jnp.zeros((1024, 16), jnp.float32)
    zeros1d = jnp.zeros((2048,), jnp.float32)

    cl2 = _cl_from_pos(pos)
    cl = cl2.reshape(N)

    deg0, deg1, h0, h1 = _sc_preprocess(esd, cl, zeros1d)
    h0 = h0[:NC1 * HP].reshape(NC1, HP)
    h1 = h1[:NC1 * HP].reshape(NC1, HP)
    w1p = jnp.pad(W1, ((0, 0), (0, 8)))
    dinv, z = _tc1(deg0.reshape(N, 1), deg1.reshape(N, 1), x, w1p)

    stats16 = _make_stats(1)
    stats32 = _make_stats(2)
    epi_1 = _make_epilogue(16, 8, 16, residual=False, matmul=True)
    epi_mid = _make_epilogue(16, 16, 16, residual=False, matmul=True)
    epi_res = _make_epilogue(16, 16, 32, residual=True, matmul=True)
    epi_5 = _make_epilogue(32, 32, 0, residual=False, matmul=False)

    g = lambda a: a.reshape(1, -1)

    parts = _sc_aggregate(z, esd, zeros2d)
    p, st = stats16(parts, [z], dinv)
    x1, z = epi_1(p, st, g(gamma1), g(beta1), W2, dinv)
    parts = _sc_aggregate(z, esd, zeros2d)
    p, st = stats16(parts, [z], dinv)
    x2, z = epi_mid(p, st, g(gamma2), g(beta2), W3, dinv)
    parts = _sc_aggregate(z, esd, zeros2d)
    p, st = stats16(parts, [z], dinv)
    x3, z = epi_mid(p, st, g(gamma3), g(beta3), W4, dinv)
    parts = _sc_aggregate(z, esd, zeros2d)
    p, st = stats16(parts, [z], dinv)
    x4, z5a, z5b = epi_res(p, st, g(gamma4), g(beta4), W5, dinv, res=x2)
    parts_a = _sc_aggregate(z5a, esd, zeros2d)
    parts_b = _sc_aggregate(z5b, esd, zeros2d)
    p, st = stats32(list(parts_a) + list(parts_b), [z5a, z5b], dinv)
    x5 = epi_5(p, st, g(gamma5), g(beta5))
    tflat = _sc_pool_max(x5.reshape(N * 32), cl)
    tables = tflat.reshape(_NW, _TBLP)[:, :NC1 * 32].reshape(_NW, NC1, 32)
    fcw3 = fcW.reshape(NC7, 32, 2)
    return _final(h0, h1, tables, W6, g(gamma6), g(beta6), W7, g(gamma7),
                  g(beta7), fcw3)
